# baseline (speedup 1.0000x reference)
import functools

import jax
import jax.numpy as jnp
from jax import lax
from jax.experimental import pallas as pl
from jax.experimental.pallas import tpu as pltpu
from jax.experimental.pallas import tpu_sc as plsc

B, N, S, D1, D2 = 4, 8192, 2048, 64, 128
C0, C1 = 128, 128
NB1 = 512
NB2 = 1024
EPS = 1e-5

SC_CORES, SC_SUBCORES = 2, 16
NW = SC_CORES * SC_SUBCORES
PTS_PER_W = (B * N) // NW
CH = 128


def _k1_body(x1_ref, x2_ref, sq1_ref, sq2_ref, idx_ref, w_ref):
    b = pl.program_id(0)
    x1 = x1_ref[0]
    x2 = x2_ref[0]
    sq1r = sq1_ref[0]
    sq2c = sq2_ref[0]

    d = -2.0 * jax.lax.dot_general(x2, x1, (((0,), (0,)), ((), ())),
                                   preferred_element_type=jnp.float32)
    d = d + sq2c + sq1r

    iota = jax.lax.broadcasted_iota(jnp.int32, d.shape, 0)
    dm = d
    idxs, recs = [], []
    recsum = jnp.zeros((1, d.shape[1]), jnp.float32)
    for _ in range(3):
        m = jnp.min(dm, axis=0, keepdims=True)
        hit = dm == m
        i = jnp.min(jnp.where(hit, iota, S), axis=0, keepdims=True)
        sel = iota == i
        rec = 1.0 / (m + 1e-8)
        idxs.append(i)
        recs.append(rec)
        recsum = recsum + rec
        dm = jnp.where(sel, jnp.float32(jnp.inf), dm)

    inv = 1.0 / recsum
    for k in range(3):
        idx_ref[0, pl.ds(k, 1), :] = idxs[k] + b * S
        w_ref[0, pl.ds(k, 1), :] = recs[k] * inv


def _k2_sc_body(i0_hbm, i1_hbm, i2_hbm, w0_hbm, w1_hbm, w2_hbm,
                tab_hbm, itp_hbm,
                i0_v, i1_v, i2_v, w0_v, w1_v, w2_v,
                r0_v, r1_v, r2_v, out_v, sem):
    wid = lax.axis_index("s") * SC_CORES + lax.axis_index("c")
    base = wid * PTS_PER_W
    idx_hbms = (i0_hbm, i1_hbm, i2_hbm)
    w_hbms = (w0_hbm, w1_hbm, w2_hbm)
    idx_vs = (i0_v, i1_v, i2_v)
    w_vs = (w0_v, w1_v, w2_v)
    rows_vs = (r0_v, r1_v, r2_v)

    def chunk(c, _):
        pbase = base + c * CH
        for k in range(3):
            pltpu.sync_copy(idx_hbms[k].at[pl.ds(pbase, CH)], idx_vs[k])
            pltpu.sync_copy(w_hbms[k].at[pl.ds(pbase, CH)], w_vs[k])
        copies = [
            pltpu.async_copy(tab_hbm.at[idx_vs[k]], rows_vs[k], sem)
            for k in range(3)
        ]
        for cp in copies:
            cp.wait()

        def group(g, _):
            gb = g * 16
            wv = [w_vs[k][pl.ds(gb, 16)] for k in range(3)]
            for pp in range(16):
                p = gb + pp
                for dv in range(D2 // 16):
                    sl = pl.ds(dv * 16, 16)
                    acc = (r0_v[p, sl] * wv[0][pp]
                           + r1_v[p, sl] * wv[1][pp]
                           + r2_v[p, sl] * wv[2][pp])
                    out_v[p, sl] = acc
            return _

        lax.fori_loop(0, CH // 16, group, None)
        pltpu.sync_copy(out_v, itp_hbm.at[pl.ds(pbase, CH)])
        return _

    lax.fori_loop(0, PTS_PER_W // CH, chunk, None)


def _k3_body(p1_ref, itp_ref, w0a_ref, w0b_ref, b0_ref, h0_ref, s_ref, ss_ref):
    b = pl.program_id(0)
    j = pl.program_id(1)
    p1 = p1_ref[0]
    itp = itp_ref[...]
    h0 = jax.lax.dot_general(w0a_ref[...], p1, (((1,), (0,)), ((), ())),
                             preferred_element_type=jnp.float32)
    h0 = h0 + jax.lax.dot_general(w0b_ref[...], itp, (((1,), (1,)), ((), ())),
                                  preferred_element_type=jnp.float32)
    h0 = h0 + b0_ref[...]
    h0_ref[0] = h0

    @pl.when((b == 0) & (j == 0))
    def _init():
        s_ref[...] = jnp.zeros_like(s_ref)
        ss_ref[...] = jnp.zeros_like(ss_ref)

    s_ref[...] += jnp.sum(h0, axis=1, keepdims=True)
    ss_ref[...] += jnp.sum(h0 * h0, axis=1, keepdims=True)


def _k4_body(h0_ref, a_ref, c_ref, w1_ref, b1_ref, h1_ref, s_ref, ss_ref):
    b = pl.program_id(0)
    j = pl.program_id(1)
    z = jnp.maximum(h0_ref[0] * a_ref[...] + c_ref[...], 0.0)
    h1 = jax.lax.dot_general(w1_ref[...], z, (((1,), (0,)), ((), ())),
                             preferred_element_type=jnp.float32) + b1_ref[...]
    h1_ref[0] = h1

    @pl.when((b == 0) & (j == 0))
    def _init():
        s_ref[...] = jnp.zeros_like(s_ref)
        ss_ref[...] = jnp.zeros_like(ss_ref)

    s_ref[...] += jnp.sum(h1, axis=1, keepdims=True)
    ss_ref[...] += jnp.sum(h1 * h1, axis=1, keepdims=True)


def _k5_body(h1_ref, a_ref, c_ref, out_ref):
    out_ref[0] = jnp.maximum(h1_ref[0] * a_ref[...] + c_ref[...], 0.0)


def _run_topk(xyz1, xyz2):
    f32 = jnp.float32
    sq1 = jnp.sum(xyz1 * xyz1, axis=1, keepdims=True)
    sq2 = jnp.sum(xyz2 * xyz2, axis=1)[:, :, None]
    gidx, wts = pl.pallas_call(
        _k1_body,
        grid=(B, N // NB1),
        in_specs=[
            pl.BlockSpec((1, 3, NB1), lambda b, j: (b, 0, j)),
            pl.BlockSpec((1, 3, S), lambda b, j: (b, 0, 0)),
            pl.BlockSpec((1, 1, NB1), lambda b, j: (b, 0, j)),
            pl.BlockSpec((1, S, 1), lambda b, j: (b, 0, 0)),
        ],
        out_specs=[
            pl.BlockSpec((1, 3, NB1), lambda b, j: (b, 0, j)),
            pl.BlockSpec((1, 3, NB1), lambda b, j: (b, 0, j)),
        ],
        out_shape=[
            jax.ShapeDtypeStruct((B, 3, N), jnp.int32),
            jax.ShapeDtypeStruct((B, 3, N), f32),
        ],
    )(xyz1, xyz2, sq1, sq2)
    return gidx, wts


def _run_sc_interp(gidx, wts, points2):
    f32 = jnp.float32
    gidx_f = jnp.transpose(gidx, (1, 0, 2)).reshape(3, B * N)
    wts_f = jnp.transpose(wts, (1, 0, 2)).reshape(3, B * N)
    table = jnp.transpose(points2, (0, 2, 1)).reshape(B * S, D2)

    sc_gather = pl.kernel(
        _k2_sc_body,
        out_type=jax.ShapeDtypeStruct((B * N, D2), f32),
        mesh=plsc.VectorSubcoreMesh(core_axis_name="c", subcore_axis_name="s"),
        scratch_types=[
            pltpu.VMEM((CH,), jnp.int32),
            pltpu.VMEM((CH,), jnp.int32),
            pltpu.VMEM((CH,), jnp.int32),
            pltpu.VMEM((CH,), f32),
            pltpu.VMEM((CH,), f32),
            pltpu.VMEM((CH,), f32),
            pltpu.VMEM((CH, D2), f32),
            pltpu.VMEM((CH, D2), f32),
            pltpu.VMEM((CH, D2), f32),
            pltpu.VMEM((CH, D2), f32),
            pltpu.SemaphoreType.DMA,
        ],
    )
    itp = sc_gather(gidx_f[0], gidx_f[1], gidx_f[2],
                    wts_f[0], wts_f[1], wts_f[2], table)
    return itp


def kernel(xyz1, xyz2, points1, points2, w0, b0, g0, be0, w1, b1, g1, be1):
    f32 = jnp.float32
    w0a = w0[:, :D1]
    w0b = w0[:, D1:]
    col = lambda v: v.reshape(-1, 1).astype(f32)

    gidx, wts = _run_topk(xyz1, xyz2)
    itp = _run_sc_interp(gidx, wts, points2)

    h0, s0, ss0 = pl.pallas_call(
        _k3_body,
        grid=(B, N // NB1),
        in_specs=[
            pl.BlockSpec((1, D1, NB1), lambda b, j: (b, 0, j)),
            pl.BlockSpec((NB1, D2), lambda b, j: (b * (N // NB1) + j, 0)),
            pl.BlockSpec((C0, D1), lambda b, j: (0, 0)),
            pl.BlockSpec((C0, D2), lambda b, j: (0, 0)),
            pl.BlockSpec((C0, 1), lambda b, j: (0, 0)),
        ],
        out_specs=[
            pl.BlockSpec((1, C0, NB1), lambda b, j: (b, 0, j)),
            pl.BlockSpec((C0, 1), lambda b, j: (0, 0)),
            pl.BlockSpec((C0, 1), lambda b, j: (0, 0)),
        ],
        out_shape=[
            jax.ShapeDtypeStruct((B, C0, N), f32),
            jax.ShapeDtypeStruct((C0, 1), f32),
            jax.ShapeDtypeStruct((C0, 1), f32),
        ],
    )(points1, itp, w0a, w0b, col(b0))

    n = float(B * N)
    mean0 = s0 / n
    var0 = ss0 / n - mean0 * mean0
    a0 = col(g0) * jax.lax.rsqrt(var0 + EPS)
    c0 = col(be0) - mean0 * a0

    h1, s1, ss1 = pl.pallas_call(
        _k4_body,
        grid=(B, N // NB2),
        in_specs=[
            pl.BlockSpec((1, C0, NB2), lambda b, j: (b, 0, j)),
            pl.BlockSpec((C0, 1), lambda b, j: (0, 0)),
            pl.BlockSpec((C0, 1), lambda b, j: (0, 0)),
            pl.BlockSpec((C1, C0), lambda b, j: (0, 0)),
            pl.BlockSpec((C1, 1), lambda b, j: (0, 0)),
        ],
        out_specs=[
            pl.BlockSpec((1, C1, NB2), lambda b, j: (b, 0, j)),
            pl.BlockSpec((C1, 1), lambda b, j: (0, 0)),
            pl.BlockSpec((C1, 1), lambda b, j: (0, 0)),
        ],
        out_shape=[
            jax.ShapeDtypeStruct((B, C1, N), f32),
            jax.ShapeDtypeStruct((C1, 1), f32),
            jax.ShapeDtypeStruct((C1, 1), f32),
        ],
    )(h0, a0, c0, w1, col(b1))

    mean1 = s1 / n
    var1 = ss1 / n - mean1 * mean1
    a1 = col(g1) * jax.lax.rsqrt(var1 + EPS)
    c1 = col(be1) - mean1 * a1

    out = pl.pallas_call(
        _k5_body,
        grid=(B, N // NB2),
        in_specs=[
            pl.BlockSpec((1, C1, NB2), lambda b, j: (b, 0, j)),
            pl.BlockSpec((C1, 1), lambda b, j: (0, 0)),
            pl.BlockSpec((C1, 1), lambda b, j: (0, 0)),
        ],
        out_specs=pl.BlockSpec((1, C1, NB2), lambda b, j: (b, 0, j)),
        out_shape=jax.ShapeDtypeStruct((B, C1, N), f32),
    )(h1, a1, c1)

    return out


# speedup vs baseline: 1.1156x; 1.1156x over previous
import functools

import jax
import jax.numpy as jnp
from jax import lax
from jax.experimental import pallas as pl
from jax.experimental.pallas import tpu as pltpu
from jax.experimental.pallas import tpu_sc as plsc

B, N, S, D1, D2 = 4, 8192, 2048, 64, 128
C0, C1 = 128, 128
NB1 = 512
NB2 = 1024
EPS = 1e-5

SC_CORES, SC_SUBCORES = 2, 16
NW = SC_CORES * SC_SUBCORES
PTS_PER_W = (B * N) // NW
CH = 64


def _k1_body(x1_ref, x2m_ref, sq1_ref, sq2_ref, idx_ref, w_ref):
    b = pl.program_id(0)
    x1 = x1_ref[0]
    x2m = x2m_ref[0]
    sq1r = sq1_ref[0]
    sq2c = sq2_ref[0]

    dm = jax.lax.dot_general(x2m, x1, (((0,), (0,)), ((), ())),
                             preferred_element_type=jnp.float32) + sq2c

    iota = jax.lax.broadcasted_iota(jnp.int32, dm.shape, 0)
    idxs, recs = [], []
    recsum = jnp.zeros((1, dm.shape[1]), jnp.float32)
    for k in range(3):
        m = jnp.min(dm, axis=0, keepdims=True)
        i = jnp.min(jnp.where(dm == m, iota, S), axis=0, keepdims=True)
        rec = 1.0 / ((m + sq1r) + 1e-8)
        idxs.append(i)
        recs.append(rec)
        recsum = recsum + rec
        if k < 2:
            dm = jnp.where(iota == i, jnp.float32(jnp.inf), dm)

    inv = 1.0 / recsum
    for k in range(3):
        idx_ref[pl.ds(k, 1), :] = idxs[k] + b * S
        w_ref[pl.ds(k, 1), :] = recs[k] * inv


def _k2_sc_body(i0_hbm, i1_hbm, i2_hbm, w0_hbm, w1_hbm, w2_hbm,
                tab_hbm, itp_hbm,
                i0_v, i1_v, i2_v, w0_v, w1_v, w2_v,
                a0_v, a1_v, a2_v, b0_v, b1_v, b2_v, out_v, sem_a, sem_b):
    wid = lax.axis_index("s") * SC_CORES + lax.axis_index("c")
    base = wid * PTS_PER_W
    idx_vs = (i0_v, i1_v, i2_v)
    w_vs = (w0_v, w1_v, w2_v)
    bufs = ((a0_v, a1_v, a2_v), (b0_v, b1_v, b2_v))
    sems = (sem_a, sem_b)
    nch = PTS_PER_W // CH

    for k, h in enumerate((i0_hbm, i1_hbm, i2_hbm)):
        pltpu.sync_copy(h.at[pl.ds(base, PTS_PER_W)], idx_vs[k])
    for k, h in enumerate((w0_hbm, w1_hbm, w2_hbm)):
        pltpu.sync_copy(h.at[pl.ds(base, PTS_PER_W)], w_vs[k])

    def fire(c, par):
        for k in range(3):
            pltpu.async_copy(
                tab_hbm.at[idx_vs[k].at[pl.ds(c * CH, CH)]], bufs[par][k],
                sems[par])

    def drain(c, par):
        for k in range(3):
            pltpu.make_async_copy(
                tab_hbm.at[idx_vs[k].at[pl.ds(c * CH, CH)]], bufs[par][k],
                sems[par]).wait()

    def compute(c, par):
        rows = bufs[par]

        def group(g, _):
            gb = g * 16
            wv = [w_vs[k][pl.ds(c * CH + gb, 16)] for k in range(3)]
            for pp in range(16):
                p = gb + pp
                for dv in range(D2 // 16):
                    sl = pl.ds(dv * 16, 16)
                    out_v[p, sl] = (rows[0][p, sl] * wv[0][pp]
                                    + rows[1][p, sl] * wv[1][pp]
                                    + rows[2][p, sl] * wv[2][pp])
            return _

        lax.fori_loop(0, CH // 16, group, None)
        pltpu.sync_copy(out_v, itp_hbm.at[pl.ds(base + c * CH, CH)])

    fire(0, 0)

    def pair(t, _):
        c0 = 2 * t
        fire(c0 + 1, 1)
        drain(c0, 0)
        compute(c0, 0)

        @pl.when(t + 1 < nch // 2)
        def _():
            fire(c0 + 2, 0)

        drain(c0 + 1, 1)
        compute(c0 + 1, 1)
        return _

    lax.fori_loop(0, nch // 2, pair, None)


def _k3_body(p1_ref, itp_ref, w0a_ref, w0b_ref, b0_ref, h0_ref, s_ref, ss_ref):
    b = pl.program_id(0)
    j = pl.program_id(1)
    p1 = p1_ref[0]
    itp = itp_ref[...]
    h0 = jax.lax.dot_general(w0a_ref[...], p1, (((1,), (0,)), ((), ())),
                             preferred_element_type=jnp.float32)
    h0 = h0 + jax.lax.dot_general(w0b_ref[...], itp, (((1,), (1,)), ((), ())),
                                  preferred_element_type=jnp.float32)
    h0 = h0 + b0_ref[...]
    h0_ref[0] = h0

    @pl.when((b == 0) & (j == 0))
    def _init():
        s_ref[...] = jnp.zeros_like(s_ref)
        ss_ref[...] = jnp.zeros_like(ss_ref)

    s_ref[...] += jnp.sum(h0, axis=1, keepdims=True)
    ss_ref[...] += jnp.sum(h0 * h0, axis=1, keepdims=True)


def _k4_body(h0_ref, a_ref, c_ref, w1_ref, b1_ref, h1_ref, s_ref, ss_ref):
    b = pl.program_id(0)
    j = pl.program_id(1)
    z = jnp.maximum(h0_ref[0] * a_ref[...] + c_ref[...], 0.0)
    h1 = jax.lax.dot_general(w1_ref[...], z, (((1,), (0,)), ((), ())),
                             preferred_element_type=jnp.float32) + b1_ref[...]
    h1_ref[0] = h1

    @pl.when((b == 0) & (j == 0))
    def _init():
        s_ref[...] = jnp.zeros_like(s_ref)
        ss_ref[...] = jnp.zeros_like(ss_ref)

    s_ref[...] += jnp.sum(h1, axis=1, keepdims=True)
    ss_ref[...] += jnp.sum(h1 * h1, axis=1, keepdims=True)


def _k5_body(h1_ref, a_ref, c_ref, out_ref):
    out_ref[0] = jnp.maximum(h1_ref[0] * a_ref[...] + c_ref[...], 0.0)


def _run_topk(xyz1, xyz2):
    f32 = jnp.float32
    x2m = xyz2 * jnp.float32(-2.0)
    sq1 = jnp.sum(xyz1 * xyz1, axis=1, keepdims=True)
    sq2 = jnp.sum(xyz2 * xyz2, axis=1)[:, :, None]
    nbs = N // NB1
    gidx, wts = pl.pallas_call(
        _k1_body,
        grid=(B, nbs),
        in_specs=[
            pl.BlockSpec((1, 3, NB1), lambda b, j: (b, 0, j)),
            pl.BlockSpec((1, 3, S), lambda b, j: (b, 0, 0)),
            pl.BlockSpec((1, 1, NB1), lambda b, j: (b, 0, j)),
            pl.BlockSpec((1, S, 1), lambda b, j: (b, 0, 0)),
        ],
        out_specs=[
            pl.BlockSpec((3, NB1), lambda b, j: (0, b * (N // NB1) + j)),
            pl.BlockSpec((3, NB1), lambda b, j: (0, b * (N // NB1) + j)),
        ],
        out_shape=[
            jax.ShapeDtypeStruct((3, B * N), jnp.int32),
            jax.ShapeDtypeStruct((3, B * N), f32),
        ],
    )(xyz1, x2m, sq1, sq2)
    return gidx, wts


def _run_sc_interp(gidx_f, wts_f, points2):
    f32 = jnp.float32
    table = jnp.transpose(points2, (0, 2, 1)).reshape(B * S, D2)

    sc_gather = pl.kernel(
        _k2_sc_body,
        out_type=jax.ShapeDtypeStruct((B * N, D2), f32),
        mesh=plsc.VectorSubcoreMesh(core_axis_name="c", subcore_axis_name="s"),
        scratch_types=[
            pltpu.VMEM((PTS_PER_W,), jnp.int32),
            pltpu.VMEM((PTS_PER_W,), jnp.int32),
            pltpu.VMEM((PTS_PER_W,), jnp.int32),
            pltpu.VMEM((PTS_PER_W,), f32),
            pltpu.VMEM((PTS_PER_W,), f32),
            pltpu.VMEM((PTS_PER_W,), f32),
            pltpu.VMEM((CH, D2), f32),
            pltpu.VMEM((CH, D2), f32),
            pltpu.VMEM((CH, D2), f32),
            pltpu.VMEM((CH, D2), f32),
            pltpu.VMEM((CH, D2), f32),
            pltpu.VMEM((CH, D2), f32),
            pltpu.VMEM((CH, D2), f32),
            pltpu.SemaphoreType.DMA,
            pltpu.SemaphoreType.DMA,
        ],
    )
    itp = sc_gather(gidx_f[0], gidx_f[1], gidx_f[2],
                    wts_f[0], wts_f[1], wts_f[2], table)
    return itp


def kernel(xyz1, xyz2, points1, points2, w0, b0, g0, be0, w1, b1, g1, be1):
    f32 = jnp.float32
    w0a = w0[:, :D1]
    w0b = w0[:, D1:]
    col = lambda v: v.reshape(-1, 1).astype(f32)

    gidx_f, wts_f = _run_topk(xyz1, xyz2)
    itp = _run_sc_interp(gidx_f, wts_f, points2)

    h0, s0, ss0 = pl.pallas_call(
        _k3_body,
        grid=(B, N // NB1),
        in_specs=[
            pl.BlockSpec((1, D1, NB1), lambda b, j: (b, 0, j)),
            pl.BlockSpec((NB1, D2), lambda b, j: (b * (N // NB1) + j, 0)),
            pl.BlockSpec((C0, D1), lambda b, j: (0, 0)),
            pl.BlockSpec((C0, D2), lambda b, j: (0, 0)),
            pl.BlockSpec((C0, 1), lambda b, j: (0, 0)),
        ],
        out_specs=[
            pl.BlockSpec((1, C0, NB1), lambda b, j: (b, 0, j)),
            pl.BlockSpec((C0, 1), lambda b, j: (0, 0)),
            pl.BlockSpec((C0, 1), lambda b, j: (0, 0)),
        ],
        out_shape=[
            jax.ShapeDtypeStruct((B, C0, N), f32),
            jax.ShapeDtypeStruct((C0, 1), f32),
            jax.ShapeDtypeStruct((C0, 1), f32),
        ],
    )(points1, itp, w0a, w0b, col(b0))

    n = float(B * N)
    mean0 = s0 / n
    var0 = ss0 / n - mean0 * mean0
    a0 = col(g0) * jax.lax.rsqrt(var0 + EPS)
    c0 = col(be0) - mean0 * a0

    h1, s1, ss1 = pl.pallas_call(
        _k4_body,
        grid=(B, N // NB2),
        in_specs=[
            pl.BlockSpec((1, C0, NB2), lambda b, j: (b, 0, j)),
            pl.BlockSpec((C0, 1), lambda b, j: (0, 0)),
            pl.BlockSpec((C0, 1), lambda b, j: (0, 0)),
            pl.BlockSpec((C1, C0), lambda b, j: (0, 0)),
            pl.BlockSpec((C1, 1), lambda b, j: (0, 0)),
        ],
        out_specs=[
            pl.BlockSpec((1, C1, NB2), lambda b, j: (b, 0, j)),
            pl.BlockSpec((C1, 1), lambda b, j: (0, 0)),
            pl.BlockSpec((C1, 1), lambda b, j: (0, 0)),
        ],
        out_shape=[
            jax.ShapeDtypeStruct((B, C1, N), f32),
            jax.ShapeDtypeStruct((C1, 1), f32),
            jax.ShapeDtypeStruct((C1, 1), f32),
        ],
    )(h0, a0, c0, w1, col(b1))

    mean1 = s1 / n
    var1 = ss1 / n - mean1 * mean1
    a1 = col(g1) * jax.lax.rsqrt(var1 + EPS)
    c1 = col(be1) - mean1 * a1

    out = pl.pallas_call(
        _k5_body,
        grid=(B, N // NB2),
        in_specs=[
            pl.BlockSpec((1, C1, NB2), lambda b, j: (b, 0, j)),
            pl.BlockSpec((C1, 1), lambda b, j: (0, 0)),
            pl.BlockSpec((C1, 1), lambda b, j: (0, 0)),
        ],
        out_specs=pl.BlockSpec((1, C1, NB2), lambda b, j: (b, 0, j)),
        out_shape=jax.ShapeDtypeStruct((B, C1, N), f32),
    )(h1, a1, c1)

    return out


# speedup vs baseline: 1.2574x; 1.1272x over previous
import functools

import jax
import jax.numpy as jnp
from jax import lax
from jax.experimental import pallas as pl
from jax.experimental.pallas import tpu as pltpu
from jax.experimental.pallas import tpu_sc as plsc

B, N, S, D1, D2 = 4, 8192, 2048, 64, 128
C0, C1 = 128, 128
NB1 = 512
NBF = 1024
EPS = 1e-5

SC_CORES, SC_SUBCORES = 2, 16
NW = SC_CORES * SC_SUBCORES
PTS_PER_W = (B * N) // NW
CH = 64


def _k1_body(x1_ref, x2m_ref, sq1_ref, sq2_ref, idx_ref, w_ref):
    b = pl.program_id(0)
    x1 = x1_ref[0]
    x2m = x2m_ref[0]
    sq1r = sq1_ref[0]
    sq2c = sq2_ref[0]

    dm = jax.lax.dot_general(x2m, x1, (((0,), (0,)), ((), ())),
                             preferred_element_type=jnp.float32) + sq2c

    iota = jax.lax.broadcasted_iota(jnp.int32, dm.shape, 0)
    idxs, recs = [], []
    recsum = jnp.zeros((1, dm.shape[1]), jnp.float32)
    for k in range(3):
        m = jnp.min(dm, axis=0, keepdims=True)
        i = jnp.min(jnp.where(dm == m, iota, S), axis=0, keepdims=True)
        rec = 1.0 / ((m + sq1r) + 1e-8)
        idxs.append(i)
        recs.append(rec)
        recsum = recsum + rec
        if k < 2:
            dm = jnp.where(iota == i, jnp.float32(jnp.inf), dm)

    inv = 1.0 / recsum
    for k in range(3):
        idx_ref[pl.ds(k, 1), :] = idxs[k] + b * S
        w_ref[pl.ds(k, 1), :] = recs[k] * inv


def _k2_sc_body(i0_hbm, i1_hbm, i2_hbm, w0_hbm, w1_hbm, w2_hbm,
                tab_hbm, itp_hbm,
                i0_v, i1_v, i2_v, w0_v, w1_v, w2_v,
                a0_v, a1_v, a2_v, b0_v, b1_v, b2_v, out_v, sem_a, sem_b):
    wid = lax.axis_index("s") * SC_CORES + lax.axis_index("c")
    base = wid * PTS_PER_W
    idx_vs = (i0_v, i1_v, i2_v)
    w_vs = (w0_v, w1_v, w2_v)
    bufs = ((a0_v, a1_v, a2_v), (b0_v, b1_v, b2_v))
    sems = (sem_a, sem_b)
    nch = PTS_PER_W // CH

    for k, h in enumerate((i0_hbm, i1_hbm, i2_hbm)):
        pltpu.sync_copy(h.at[pl.ds(base, PTS_PER_W)], idx_vs[k])
    for k, h in enumerate((w0_hbm, w1_hbm, w2_hbm)):
        pltpu.sync_copy(h.at[pl.ds(base, PTS_PER_W)], w_vs[k])

    def fire(c, par):
        for k in range(3):
            pltpu.async_copy(
                tab_hbm.at[idx_vs[k].at[pl.ds(c * CH, CH)]], bufs[par][k],
                sems[par])

    def drain(c, par):
        for k in range(3):
            pltpu.make_async_copy(
                tab_hbm.at[idx_vs[k].at[pl.ds(c * CH, CH)]], bufs[par][k],
                sems[par]).wait()

    def compute(c, par):
        rows = bufs[par]

        def group(g, _):
            gb = g * 16
            wv = [w_vs[k][pl.ds(c * CH + gb, 16)] for k in range(3)]
            for pp in range(16):
                p = gb + pp
                for dv in range(D2 // 16):
                    sl = pl.ds(dv * 16, 16)
                    out_v[p, sl] = (rows[0][p, sl] * wv[0][pp]
                                    + rows[1][p, sl] * wv[1][pp]
                                    + rows[2][p, sl] * wv[2][pp])
            return _

        lax.fori_loop(0, CH // 16, group, None)
        pltpu.sync_copy(out_v, itp_hbm.at[pl.ds(base + c * CH, CH)])

    fire(0, 0)

    def pair(t, _):
        c0 = 2 * t
        fire(c0 + 1, 1)
        drain(c0, 0)
        compute(c0, 0)

        @pl.when(t + 1 < nch // 2)
        def _():
            fire(c0 + 2, 0)

        drain(c0 + 1, 1)
        compute(c0 + 1, 1)
        return _

    lax.fori_loop(0, nch // 2, pair, None)


def _mlp_body(p1_ref, itp_ref, w0a_ref, w0b_ref, b0_ref, w1_ref, b1_ref,
              g0_ref, be0_ref, g1_ref, be1_ref, out_ref,
              hbuf, s0_ref, ss0_ref, s1_ref, ss1_ref):
    ph = pl.program_id(0)
    b = pl.program_id(1)
    j = pl.program_id(2)
    n = jnp.float32(B * N)
    sl = pl.ds(j * NBF, NBF)

    @pl.when(ph == 0)
    def _phase0():
        @pl.when((b == 0) & (j == 0))
        def _init():
            s0_ref[...] = jnp.zeros_like(s0_ref)
            ss0_ref[...] = jnp.zeros_like(ss0_ref)

        p1 = p1_ref[0]
        itp = itp_ref[...]
        h0 = jax.lax.dot_general(w0a_ref[...], p1, (((1,), (0,)), ((), ())),
                                 preferred_element_type=jnp.float32)
        h0 = h0 + jax.lax.dot_general(
            w0b_ref[...], itp, (((1,), (1,)), ((), ())),
            preferred_element_type=jnp.float32)
        h0 = h0 + b0_ref[...]
        hbuf[b, :, sl] = h0
        s0_ref[...] += jnp.sum(h0, axis=1, keepdims=True)
        ss0_ref[...] += jnp.sum(h0 * h0, axis=1, keepdims=True)

    @pl.when(ph == 1)
    def _phase1():
        @pl.when((b == 0) & (j == 0))
        def _init():
            s1_ref[...] = jnp.zeros_like(s1_ref)
            ss1_ref[...] = jnp.zeros_like(ss1_ref)

        mean0 = s0_ref[...] / n
        var0 = ss0_ref[...] / n - mean0 * mean0
        a0 = g0_ref[...] * jax.lax.rsqrt(var0 + EPS)
        c0 = be0_ref[...] - mean0 * a0
        z = jnp.maximum(hbuf[b, :, sl] * a0 + c0, 0.0)
        h1 = jax.lax.dot_general(w1_ref[...], z, (((1,), (0,)), ((), ())),
                                 preferred_element_type=jnp.float32)
        h1 = h1 + b1_ref[...]
        hbuf[b, :, sl] = h1
        s1_ref[...] += jnp.sum(h1, axis=1, keepdims=True)
        ss1_ref[...] += jnp.sum(h1 * h1, axis=1, keepdims=True)

    @pl.when(ph == 2)
    def _phase2():
        mean1 = s1_ref[...] / n
        var1 = ss1_ref[...] / n - mean1 * mean1
        a1 = g1_ref[...] * jax.lax.rsqrt(var1 + EPS)
        c1 = be1_ref[...] - mean1 * a1
        out_ref[0] = jnp.maximum(hbuf[b, :, sl] * a1 + c1, 0.0)


def _run_topk(xyz1, xyz2):
    f32 = jnp.float32
    x2m = xyz2 * jnp.float32(-2.0)
    sq1 = jnp.sum(xyz1 * xyz1, axis=1, keepdims=True)
    sq2 = jnp.sum(xyz2 * xyz2, axis=1)[:, :, None]
    nbs = N // NB1
    gidx, wts = pl.pallas_call(
        _k1_body,
        grid=(B, nbs),
        in_specs=[
            pl.BlockSpec((1, 3, NB1), lambda b, j: (b, 0, j)),
            pl.BlockSpec((1, 3, S), lambda b, j: (b, 0, 0)),
            pl.BlockSpec((1, 1, NB1), lambda b, j: (b, 0, j)),
            pl.BlockSpec((1, S, 1), lambda b, j: (b, 0, 0)),
        ],
        out_specs=[
            pl.BlockSpec((3, NB1), lambda b, j: (0, b * (N // NB1) + j)),
            pl.BlockSpec((3, NB1), lambda b, j: (0, b * (N // NB1) + j)),
        ],
        out_shape=[
            jax.ShapeDtypeStruct((3, B * N), jnp.int32),
            jax.ShapeDtypeStruct((3, B * N), f32),
        ],
    )(xyz1, x2m, sq1, sq2)
    return gidx, wts


def _run_sc_interp(gidx_f, wts_f, points2):
    f32 = jnp.float32
    table = jnp.transpose(points2, (0, 2, 1)).reshape(B * S, D2)

    sc_gather = pl.kernel(
        _k2_sc_body,
        out_type=jax.ShapeDtypeStruct((B * N, D2), f32),
        mesh=plsc.VectorSubcoreMesh(core_axis_name="c", subcore_axis_name="s"),
        scratch_types=[
            pltpu.VMEM((PTS_PER_W,), jnp.int32),
            pltpu.VMEM((PTS_PER_W,), jnp.int32),
            pltpu.VMEM((PTS_PER_W,), jnp.int32),
            pltpu.VMEM((PTS_PER_W,), f32),
            pltpu.VMEM((PTS_PER_W,), f32),
            pltpu.VMEM((PTS_PER_W,), f32),
            pltpu.VMEM((CH, D2), f32),
            pltpu.VMEM((CH, D2), f32),
            pltpu.VMEM((CH, D2), f32),
            pltpu.VMEM((CH, D2), f32),
            pltpu.VMEM((CH, D2), f32),
            pltpu.VMEM((CH, D2), f32),
            pltpu.VMEM((CH, D2), f32),
            pltpu.SemaphoreType.DMA,
            pltpu.SemaphoreType.DMA,
        ],
    )
    itp = sc_gather(gidx_f[0], gidx_f[1], gidx_f[2],
                    wts_f[0], wts_f[1], wts_f[2], table)
    return itp


def kernel(xyz1, xyz2, points1, points2, w0, b0, g0, be0, w1, b1, g1, be1):
    f32 = jnp.float32
    w0a = w0[:, :D1]
    w0b = w0[:, D1:]
    col = lambda v: v.reshape(-1, 1).astype(f32)

    gidx_f, wts_f = _run_topk(xyz1, xyz2)
    itp = _run_sc_interp(gidx_f, wts_f, points2)

    nj = N // NBF
    out = pl.pallas_call(
        _mlp_body,
        grid=(3, B, nj),
        in_specs=[
            pl.BlockSpec((1, D1, NBF),
                         lambda ph, b, j: (jnp.where(ph == 0, b, 0), 0,
                                           jnp.where(ph == 0, j, 0))),
            pl.BlockSpec((NBF, D2),
                         lambda ph, b, j: (jnp.where(ph == 0, b * nj + j, 0),
                                           0)),
            pl.BlockSpec((C0, D1), lambda ph, b, j: (0, 0)),
            pl.BlockSpec((C0, D2), lambda ph, b, j: (0, 0)),
            pl.BlockSpec((C0, 1), lambda ph, b, j: (0, 0)),
            pl.BlockSpec((C1, C0), lambda ph, b, j: (0, 0)),
            pl.BlockSpec((C1, 1), lambda ph, b, j: (0, 0)),
            pl.BlockSpec((C0, 1), lambda ph, b, j: (0, 0)),
            pl.BlockSpec((C0, 1), lambda ph, b, j: (0, 0)),
            pl.BlockSpec((C1, 1), lambda ph, b, j: (0, 0)),
            pl.BlockSpec((C1, 1), lambda ph, b, j: (0, 0)),
        ],
        out_specs=pl.BlockSpec(
            (1, C1, NBF),
            lambda ph, b, j: (jnp.where(ph == 2, b, 0), 0,
                              jnp.where(ph == 2, j, 0))),
        out_shape=jax.ShapeDtypeStruct((B, C1, N), f32),
        scratch_shapes=[
            pltpu.VMEM((B, C0, N), f32),
            pltpu.VMEM((C0, 1), f32),
            pltpu.VMEM((C0, 1), f32),
            pltpu.VMEM((C1, 1), f32),
            pltpu.VMEM((C1, 1), f32),
        ],
    )(points1, itp, w0a, w0b, col(b0), w1, col(b1),
      col(g0), col(be0), col(g1), col(be1))

    return out


# speedup vs baseline: 1.3585x; 1.0804x over previous
import functools

import jax
import jax.numpy as jnp
from jax import lax
from jax.experimental import pallas as pl
from jax.experimental.pallas import tpu as pltpu
from jax.experimental.pallas import tpu_sc as plsc

B, N, S, D1, D2 = 4, 8192, 2048, 64, 128
C0, C1 = 128, 128
NB1 = 1024
NBF = 1024
EPS = 1e-5

SC_CORES, SC_SUBCORES = 2, 16
NW = SC_CORES * SC_SUBCORES
PTS_PER_W = (B * N) // NW
CH = 64


def _k1_body(x1_ref, x2m_ref, sq1_ref, sq2_ref, idx_ref, w_ref):
    b = pl.program_id(0)
    x1 = x1_ref[0]
    x2m = x2m_ref[0]
    sq1r = sq1_ref[0]
    sq2c = sq2_ref[0]

    dm = jax.lax.dot_general(x2m, x1, (((0,), (0,)), ((), ())),
                             preferred_element_type=jnp.float32) + sq2c

    iota = jax.lax.broadcasted_iota(jnp.int32, dm.shape, 0)
    idxs, recs = [], []
    recsum = jnp.zeros((1, dm.shape[1]), jnp.float32)
    for k in range(3):
        m = jnp.min(dm, axis=0, keepdims=True)
        i = jnp.min(jnp.where(dm == m, iota, S), axis=0, keepdims=True)
        rec = 1.0 / ((m + sq1r) + 1e-8)
        idxs.append(i)
        recs.append(rec)
        recsum = recsum + rec
        if k < 2:
            dm = jnp.where(iota == i, jnp.float32(jnp.inf), dm)

    inv = 1.0 / recsum
    for k in range(3):
        idx_ref[pl.ds(k, 1), :] = idxs[k] + b * S
        w_ref[pl.ds(k, 1), :] = recs[k] * inv


def _k2_sc_body(i0_hbm, i1_hbm, i2_hbm, w0_hbm, w1_hbm, w2_hbm,
                tab_hbm, itp_hbm,
                i0_v, i1_v, i2_v, w0_v, w1_v, w2_v,
                a0_v, a1_v, a2_v, b0_v, b1_v, b2_v, out_v, sem_a, sem_b):
    wid = lax.axis_index("s") * SC_CORES + lax.axis_index("c")
    base = wid * PTS_PER_W
    idx_vs = (i0_v, i1_v, i2_v)
    w_vs = (w0_v, w1_v, w2_v)
    bufs = ((a0_v, a1_v, a2_v), (b0_v, b1_v, b2_v))
    sems = (sem_a, sem_b)
    nch = PTS_PER_W // CH

    for k, h in enumerate((i0_hbm, i1_hbm, i2_hbm)):
        pltpu.sync_copy(h.at[pl.ds(base, PTS_PER_W)], idx_vs[k])
    for k, h in enumerate((w0_hbm, w1_hbm, w2_hbm)):
        pltpu.sync_copy(h.at[pl.ds(base, PTS_PER_W)], w_vs[k])

    def fire(c, par):
        for k in range(3):
            pltpu.async_copy(
                tab_hbm.at[idx_vs[k].at[pl.ds(c * CH, CH)]], bufs[par][k],
                sems[par])

    def drain(c, par):
        for k in range(3):
            pltpu.make_async_copy(
                tab_hbm.at[idx_vs[k].at[pl.ds(c * CH, CH)]], bufs[par][k],
                sems[par]).wait()

    def compute(c, par):
        rows = bufs[par]

        def group(g, _):
            gb = g * 16
            wv = [w_vs[k][pl.ds(c * CH + gb, 16)] for k in range(3)]
            for pp in range(16):
                p = gb + pp
                for dv in range(D2 // 16):
                    sl = pl.ds(dv * 16, 16)
                    out_v[p, sl] = (rows[0][p, sl] * wv[0][pp]
                                    + rows[1][p, sl] * wv[1][pp]
                                    + rows[2][p, sl] * wv[2][pp])
            return _

        lax.fori_loop(0, CH // 16, group, None)
        pltpu.sync_copy(out_v, itp_hbm.at[pl.ds(base + c * CH, CH)])

    fire(0, 0)

    def pair(t, _):
        c0 = 2 * t
        fire(c0 + 1, 1)
        drain(c0, 0)
        compute(c0, 0)

        @pl.when(t + 1 < nch // 2)
        def _():
            fire(c0 + 2, 0)

        drain(c0 + 1, 1)
        compute(c0 + 1, 1)
        return _

    lax.fori_loop(0, nch // 2, pair, None)


def _mlp_body(p1_ref, itp_ref, w0a_ref, w0b_ref, b0_ref, w1_ref, b1_ref,
              g0_ref, be0_ref, g1_ref, be1_ref, out_ref,
              hbuf, s0_ref, ss0_ref, s1_ref, ss1_ref):
    ph = pl.program_id(0)
    b = pl.program_id(1)
    j = pl.program_id(2)
    n = jnp.float32(B * N)
    sl = pl.ds(j * NBF, NBF)

    @pl.when(ph == 0)
    def _phase0():
        @pl.when((b == 0) & (j == 0))
        def _init():
            s0_ref[...] = jnp.zeros_like(s0_ref)
            ss0_ref[...] = jnp.zeros_like(ss0_ref)

        p1 = p1_ref[0]
        itp = itp_ref[...]
        h0 = jax.lax.dot_general(w0a_ref[...], p1, (((1,), (0,)), ((), ())),
                                 preferred_element_type=jnp.float32)
        h0 = h0 + jax.lax.dot_general(
            w0b_ref[...], itp, (((1,), (1,)), ((), ())),
            preferred_element_type=jnp.float32)
        h0 = h0 + b0_ref[...]
        hbuf[b, :, sl] = h0
        s0_ref[...] += jnp.sum(h0, axis=1, keepdims=True)
        ss0_ref[...] += jnp.sum(h0 * h0, axis=1, keepdims=True)

    @pl.when(ph == 1)
    def _phase1():
        @pl.when((b == 0) & (j == 0))
        def _init():
            s1_ref[...] = jnp.zeros_like(s1_ref)
            ss1_ref[...] = jnp.zeros_like(ss1_ref)

        mean0 = s0_ref[...] / n
        var0 = ss0_ref[...] / n - mean0 * mean0
        a0 = g0_ref[...] * jax.lax.rsqrt(var0 + EPS)
        c0 = be0_ref[...] - mean0 * a0
        z = jnp.maximum(hbuf[b, :, sl] * a0 + c0, 0.0)
        h1 = jax.lax.dot_general(w1_ref[...], z, (((1,), (0,)), ((), ())),
                                 preferred_element_type=jnp.float32)
        h1 = h1 + b1_ref[...]
        hbuf[b, :, sl] = h1
        s1_ref[...] += jnp.sum(h1, axis=1, keepdims=True)
        ss1_ref[...] += jnp.sum(h1 * h1, axis=1, keepdims=True)

    @pl.when(ph == 2)
    def _phase2():
        mean1 = s1_ref[...] / n
        var1 = ss1_ref[...] / n - mean1 * mean1
        a1 = g1_ref[...] * jax.lax.rsqrt(var1 + EPS)
        c1 = be1_ref[...] - mean1 * a1
        out_ref[0] = jnp.maximum(hbuf[b, :, sl] * a1 + c1, 0.0)


def _run_topk(xyz1, xyz2):
    f32 = jnp.float32
    x2m = xyz2 * jnp.float32(-2.0)
    sq1 = jnp.sum(xyz1 * xyz1, axis=1, keepdims=True)
    sq2 = jnp.sum(xyz2 * xyz2, axis=1)[:, :, None]
    nbs = N // NB1
    gidx, wts = pl.pallas_call(
        _k1_body,
        grid=(B, nbs),
        in_specs=[
            pl.BlockSpec((1, 3, NB1), lambda b, j: (b, 0, j)),
            pl.BlockSpec((1, 3, S), lambda b, j: (b, 0, 0)),
            pl.BlockSpec((1, 1, NB1), lambda b, j: (b, 0, j)),
            pl.BlockSpec((1, S, 1), lambda b, j: (b, 0, 0)),
        ],
        out_specs=[
            pl.BlockSpec((3, NB1), lambda b, j: (0, b * (N // NB1) + j)),
            pl.BlockSpec((3, NB1), lambda b, j: (0, b * (N // NB1) + j)),
        ],
        out_shape=[
            jax.ShapeDtypeStruct((3, B * N), jnp.int32),
            jax.ShapeDtypeStruct((3, B * N), f32),
        ],
    )(xyz1, x2m, sq1, sq2)
    return gidx, wts


def _run_sc_interp(gidx_f, wts_f, points2):
    f32 = jnp.float32
    table = jnp.transpose(points2, (0, 2, 1)).reshape(B * S, D2)

    sc_gather = pl.kernel(
        _k2_sc_body,
        out_type=jax.ShapeDtypeStruct((B * N, D2), f32),
        mesh=plsc.VectorSubcoreMesh(core_axis_name="c", subcore_axis_name="s"),
        scratch_types=[
            pltpu.VMEM((PTS_PER_W,), jnp.int32),
            pltpu.VMEM((PTS_PER_W,), jnp.int32),
            pltpu.VMEM((PTS_PER_W,), jnp.int32),
            pltpu.VMEM((PTS_PER_W,), f32),
            pltpu.VMEM((PTS_PER_W,), f32),
            pltpu.VMEM((PTS_PER_W,), f32),
            pltpu.VMEM((CH, D2), f32),
            pltpu.VMEM((CH, D2), f32),
            pltpu.VMEM((CH, D2), f32),
            pltpu.VMEM((CH, D2), f32),
            pltpu.VMEM((CH, D2), f32),
            pltpu.VMEM((CH, D2), f32),
            pltpu.VMEM((CH, D2), f32),
            pltpu.SemaphoreType.DMA,
            pltpu.SemaphoreType.DMA,
        ],
    )
    itp = sc_gather(gidx_f[0], gidx_f[1], gidx_f[2],
                    wts_f[0], wts_f[1], wts_f[2], table)
    return itp


def kernel(xyz1, xyz2, points1, points2, w0, b0, g0, be0, w1, b1, g1, be1):
    f32 = jnp.float32
    w0a = w0[:, :D1]
    w0b = w0[:, D1:]
    col = lambda v: v.reshape(-1, 1).astype(f32)

    gidx_f, wts_f = _run_topk(xyz1, xyz2)
    itp = _run_sc_interp(gidx_f, wts_f, points2)

    nj = N // NBF
    out = pl.pallas_call(
        _mlp_body,
        grid=(3, B, nj),
        in_specs=[
            pl.BlockSpec((1, D1, NBF),
                         lambda ph, b, j: (jnp.where(ph == 0, b, 0), 0,
                                           jnp.where(ph == 0, j, 0))),
            pl.BlockSpec((NBF, D2),
                         lambda ph, b, j: (jnp.where(ph == 0, b * nj + j, 0),
                                           0)),
            pl.BlockSpec((C0, D1), lambda ph, b, j: (0, 0)),
            pl.BlockSpec((C0, D2), lambda ph, b, j: (0, 0)),
            pl.BlockSpec((C0, 1), lambda ph, b, j: (0, 0)),
            pl.BlockSpec((C1, C0), lambda ph, b, j: (0, 0)),
            pl.BlockSpec((C1, 1), lambda ph, b, j: (0, 0)),
            pl.BlockSpec((C0, 1), lambda ph, b, j: (0, 0)),
            pl.BlockSpec((C0, 1), lambda ph, b, j: (0, 0)),
            pl.BlockSpec((C1, 1), lambda ph, b, j: (0, 0)),
            pl.BlockSpec((C1, 1), lambda ph, b, j: (0, 0)),
        ],
        out_specs=pl.BlockSpec(
            (1, C1, NBF),
            lambda ph, b, j: (jnp.where(ph == 2, b, 0), 0,
                              jnp.where(ph == 2, j, 0))),
        out_shape=jax.ShapeDtypeStruct((B, C1, N), f32),
        scratch_shapes=[
            pltpu.VMEM((B, C0, N), f32),
            pltpu.VMEM((C0, 1), f32),
            pltpu.VMEM((C0, 1), f32),
            pltpu.VMEM((C1, 1), f32),
            pltpu.VMEM((C1, 1), f32),
        ],
    )(points1, itp, w0a, w0b, col(b0), w1, col(b1),
      col(g0), col(be0), col(g1), col(be1))

    return out


# speedup vs baseline: 1.4075x; 1.0361x over previous
import functools

import jax
import jax.numpy as jnp
from jax import lax
from jax.experimental import pallas as pl
from jax.experimental.pallas import tpu as pltpu
from jax.experimental.pallas import tpu_sc as plsc

B, N, S, D1, D2 = 4, 8192, 2048, 64, 128
C0, C1 = 128, 128
NB1 = 2048
NBF = 1024
EPS = 1e-5

SC_CORES, SC_SUBCORES = 2, 16
NW = SC_CORES * SC_SUBCORES
PTS_PER_W = (B * N) // NW
CH = 64


def _k1_body(x1_ref, x2m_ref, sq1_ref, sq2_ref, idx_ref, w_ref):
    b = pl.program_id(0)
    x1 = x1_ref[0]
    x2m = x2m_ref[0]
    sq1r = sq1_ref[0]
    sq2c = sq2_ref[0]

    dm = jax.lax.dot_general(x2m, x1, (((0,), (0,)), ((), ())),
                             preferred_element_type=jnp.float32) + sq2c

    iota = jax.lax.broadcasted_iota(jnp.int32, dm.shape, 0)
    idxs, recs = [], []
    recsum = jnp.zeros((1, dm.shape[1]), jnp.float32)
    for k in range(3):
        m = jnp.min(dm, axis=0, keepdims=True)
        i = jnp.min(jnp.where(dm == m, iota, S), axis=0, keepdims=True)
        rec = 1.0 / ((m + sq1r) + 1e-8)
        idxs.append(i)
        recs.append(rec)
        recsum = recsum + rec
        if k < 2:
            dm = jnp.where(iota == i, jnp.float32(jnp.inf), dm)

    inv = 1.0 / recsum
    for k in range(3):
        idx_ref[pl.ds(k, 1), :] = idxs[k] + b * S
        w_ref[pl.ds(k, 1), :] = recs[k] * inv


def _k2_sc_body(i0_hbm, i1_hbm, i2_hbm, w0_hbm, w1_hbm, w2_hbm,
                tab_hbm, itp_hbm,
                i0_v, i1_v, i2_v, w0_v, w1_v, w2_v,
                a0_v, a1_v, a2_v, b0_v, b1_v, b2_v, out_v, sem_a, sem_b):
    wid = lax.axis_index("s") * SC_CORES + lax.axis_index("c")
    base = wid * PTS_PER_W
    idx_vs = (i0_v, i1_v, i2_v)
    w_vs = (w0_v, w1_v, w2_v)
    bufs = ((a0_v, a1_v, a2_v), (b0_v, b1_v, b2_v))
    sems = (sem_a, sem_b)
    nch = PTS_PER_W // CH

    for k, h in enumerate((i0_hbm, i1_hbm, i2_hbm)):
        pltpu.sync_copy(h.at[pl.ds(base, PTS_PER_W)], idx_vs[k])
    for k, h in enumerate((w0_hbm, w1_hbm, w2_hbm)):
        pltpu.sync_copy(h.at[pl.ds(base, PTS_PER_W)], w_vs[k])

    def fire(c, par):
        for k in range(3):
            pltpu.async_copy(
                tab_hbm.at[idx_vs[k].at[pl.ds(c * CH, CH)]], bufs[par][k],
                sems[par])

    def drain(c, par):
        for k in range(3):
            pltpu.make_async_copy(
                tab_hbm.at[idx_vs[k].at[pl.ds(c * CH, CH)]], bufs[par][k],
                sems[par]).wait()

    def compute(c, par):
        rows = bufs[par]

        def group(g, _):
            gb = g * 16
            wv = [w_vs[k][pl.ds(c * CH + gb, 16)] for k in range(3)]
            for pp in range(16):
                p = gb + pp
                for dv in range(D2 // 16):
                    sl = pl.ds(dv * 16, 16)
                    out_v[p, sl] = (rows[0][p, sl] * wv[0][pp]
                                    + rows[1][p, sl] * wv[1][pp]
                                    + rows[2][p, sl] * wv[2][pp])
            return _

        lax.fori_loop(0, CH // 16, group, None)
        pltpu.sync_copy(out_v, itp_hbm.at[pl.ds(base + c * CH, CH)])

    fire(0, 0)

    def pair(t, _):
        c0 = 2 * t
        fire(c0 + 1, 1)
        drain(c0, 0)
        compute(c0, 0)

        @pl.when(t + 1 < nch // 2)
        def _():
            fire(c0 + 2, 0)

        drain(c0 + 1, 1)
        compute(c0 + 1, 1)
        return _

    lax.fori_loop(0, nch // 2, pair, None)


def _mlp_body(p1_ref, itp_ref, w0a_ref, w0b_ref, b0_ref, w1_ref, b1_ref,
              g0_ref, be0_ref, g1_ref, be1_ref, out_ref,
              hbuf, s0_ref, ss0_ref, s1_ref, ss1_ref):
    ph = pl.program_id(0)
    b = pl.program_id(1)
    j = pl.program_id(2)
    n = jnp.float32(B * N)
    sl = pl.ds(j * NBF, NBF)

    @pl.when(ph == 0)
    def _phase0():
        @pl.when((b == 0) & (j == 0))
        def _init():
            s0_ref[...] = jnp.zeros_like(s0_ref)
            ss0_ref[...] = jnp.zeros_like(ss0_ref)

        p1 = p1_ref[0]
        itp = itp_ref[...]
        h0 = jax.lax.dot_general(w0a_ref[...], p1, (((1,), (0,)), ((), ())),
                                 preferred_element_type=jnp.float32)
        h0 = h0 + jax.lax.dot_general(
            w0b_ref[...], itp, (((1,), (1,)), ((), ())),
            preferred_element_type=jnp.float32)
        h0 = h0 + b0_ref[...]
        hbuf[b, :, sl] = h0
        s0_ref[...] += jnp.sum(h0, axis=1, keepdims=True)
        ss0_ref[...] += jnp.sum(h0 * h0, axis=1, keepdims=True)

    @pl.when(ph == 1)
    def _phase1():
        @pl.when((b == 0) & (j == 0))
        def _init():
            s1_ref[...] = jnp.zeros_like(s1_ref)
            ss1_ref[...] = jnp.zeros_like(ss1_ref)

        mean0 = s0_ref[...] / n
        var0 = ss0_ref[...] / n - mean0 * mean0
        a0 = g0_ref[...] * jax.lax.rsqrt(var0 + EPS)
        c0 = be0_ref[...] - mean0 * a0
        z = jnp.maximum(hbuf[b, :, sl] * a0 + c0, 0.0)
        h1 = jax.lax.dot_general(w1_ref[...], z, (((1,), (0,)), ((), ())),
                                 preferred_element_type=jnp.float32)
        h1 = h1 + b1_ref[...]
        hbuf[b, :, sl] = h1
        s1_ref[...] += jnp.sum(h1, axis=1, keepdims=True)
        ss1_ref[...] += jnp.sum(h1 * h1, axis=1, keepdims=True)

    @pl.when(ph == 2)
    def _phase2():
        mean1 = s1_ref[...] / n
        var1 = ss1_ref[...] / n - mean1 * mean1
        a1 = g1_ref[...] * jax.lax.rsqrt(var1 + EPS)
        c1 = be1_ref[...] - mean1 * a1
        out_ref[0] = jnp.maximum(hbuf[b, :, sl] * a1 + c1, 0.0)


def _run_topk(xyz1, xyz2):
    f32 = jnp.float32
    x2m = xyz2 * jnp.float32(-2.0)
    sq1 = jnp.sum(xyz1 * xyz1, axis=1, keepdims=True)
    sq2 = jnp.sum(xyz2 * xyz2, axis=1)[:, :, None]
    nbs = N // NB1
    gidx, wts = pl.pallas_call(
        _k1_body,
        grid=(B, nbs),
        in_specs=[
            pl.BlockSpec((1, 3, NB1), lambda b, j: (b, 0, j)),
            pl.BlockSpec((1, 3, S), lambda b, j: (b, 0, 0)),
            pl.BlockSpec((1, 1, NB1), lambda b, j: (b, 0, j)),
            pl.BlockSpec((1, S, 1), lambda b, j: (b, 0, 0)),
        ],
        out_specs=[
            pl.BlockSpec((3, NB1), lambda b, j: (0, b * (N // NB1) + j)),
            pl.BlockSpec((3, NB1), lambda b, j: (0, b * (N // NB1) + j)),
        ],
        out_shape=[
            jax.ShapeDtypeStruct((3, B * N), jnp.int32),
            jax.ShapeDtypeStruct((3, B * N), f32),
        ],
    )(xyz1, x2m, sq1, sq2)
    return gidx, wts


def _run_sc_interp(gidx_f, wts_f, points2):
    f32 = jnp.float32
    table = jnp.transpose(points2, (0, 2, 1)).reshape(B * S, D2)

    sc_gather = pl.kernel(
        _k2_sc_body,
        out_type=jax.ShapeDtypeStruct((B * N, D2), f32),
        mesh=plsc.VectorSubcoreMesh(core_axis_name="c", subcore_axis_name="s"),
        scratch_types=[
            pltpu.VMEM((PTS_PER_W,), jnp.int32),
            pltpu.VMEM((PTS_PER_W,), jnp.int32),
            pltpu.VMEM((PTS_PER_W,), jnp.int32),
            pltpu.VMEM((PTS_PER_W,), f32),
            pltpu.VMEM((PTS_PER_W,), f32),
            pltpu.VMEM((PTS_PER_W,), f32),
            pltpu.VMEM((CH, D2), f32),
            pltpu.VMEM((CH, D2), f32),
            pltpu.VMEM((CH, D2), f32),
            pltpu.VMEM((CH, D2), f32),
            pltpu.VMEM((CH, D2), f32),
            pltpu.VMEM((CH, D2), f32),
            pltpu.VMEM((CH, D2), f32),
            pltpu.SemaphoreType.DMA,
            pltpu.SemaphoreType.DMA,
        ],
    )
    itp = sc_gather(gidx_f[0], gidx_f[1], gidx_f[2],
                    wts_f[0], wts_f[1], wts_f[2], table)
    return itp


def kernel(xyz1, xyz2, points1, points2, w0, b0, g0, be0, w1, b1, g1, be1):
    f32 = jnp.float32
    w0a = w0[:, :D1]
    w0b = w0[:, D1:]
    col = lambda v: v.reshape(-1, 1).astype(f32)

    gidx_f, wts_f = _run_topk(xyz1, xyz2)
    itp = _run_sc_interp(gidx_f, wts_f, points2)

    nj = N // NBF
    out = pl.pallas_call(
        _mlp_body,
        grid=(3, B, nj),
        in_specs=[
            pl.BlockSpec((1, D1, NBF),
                         lambda ph, b, j: (jnp.where(ph == 0, b, 0), 0,
                                           jnp.where(ph == 0, j, 0))),
            pl.BlockSpec((NBF, D2),
                         lambda ph, b, j: (jnp.where(ph == 0, b * nj + j, 0),
                                           0)),
            pl.BlockSpec((C0, D1), lambda ph, b, j: (0, 0)),
            pl.BlockSpec((C0, D2), lambda ph, b, j: (0, 0)),
            pl.BlockSpec((C0, 1), lambda ph, b, j: (0, 0)),
            pl.BlockSpec((C1, C0), lambda ph, b, j: (0, 0)),
            pl.BlockSpec((C1, 1), lambda ph, b, j: (0, 0)),
            pl.BlockSpec((C0, 1), lambda ph, b, j: (0, 0)),
            pl.BlockSpec((C0, 1), lambda ph, b, j: (0, 0)),
            pl.BlockSpec((C1, 1), lambda ph, b, j: (0, 0)),
            pl.BlockSpec((C1, 1), lambda ph, b, j: (0, 0)),
        ],
        out_specs=pl.BlockSpec(
            (1, C1, NBF),
            lambda ph, b, j: (jnp.where(ph == 2, b, 0), 0,
                              jnp.where(ph == 2, j, 0))),
        out_shape=jax.ShapeDtypeStruct((B, C1, N), f32),
        scratch_shapes=[
            pltpu.VMEM((B, C0, N), f32),
            pltpu.VMEM((C0, 1), f32),
            pltpu.VMEM((C0, 1), f32),
            pltpu.VMEM((C1, 1), f32),
            pltpu.VMEM((C1, 1), f32),
        ],
    )(points1, itp, w0a, w0b, col(b0), w1, col(b1),
      col(g0), col(be0), col(g1), col(be1))

    return out


# speedup vs baseline: 1.4184x; 1.0078x over previous
import functools

import jax
import jax.numpy as jnp
from jax import lax
from jax.experimental import pallas as pl
from jax.experimental.pallas import tpu as pltpu
from jax.experimental.pallas import tpu_sc as plsc

B, N, S, D1, D2 = 4, 8192, 2048, 64, 128
C0, C1 = 128, 128
NB1 = 4096
NBF = 1024
EPS = 1e-5

SC_CORES, SC_SUBCORES = 2, 16
NW = SC_CORES * SC_SUBCORES
PTS_PER_W = (B * N) // NW
CH = 64


def _k1_body(x1_ref, x2m_ref, sq1_ref, sq2_ref, idx_ref, w_ref):
    b = pl.program_id(0)
    x1 = x1_ref[0]
    x2m = x2m_ref[0]
    sq1r = sq1_ref[0]
    sq2c = sq2_ref[0]

    dm = jax.lax.dot_general(x2m, x1, (((0,), (0,)), ((), ())),
                             preferred_element_type=jnp.float32) + sq2c

    iota = jax.lax.broadcasted_iota(jnp.int32, dm.shape, 0)
    idxs, recs = [], []
    recsum = jnp.zeros((1, dm.shape[1]), jnp.float32)
    for k in range(3):
        m = jnp.min(dm, axis=0, keepdims=True)
        i = jnp.min(jnp.where(dm == m, iota, S), axis=0, keepdims=True)
        rec = 1.0 / ((m + sq1r) + 1e-8)
        idxs.append(i)
        recs.append(rec)
        recsum = recsum + rec
        if k < 2:
            dm = jnp.where(iota == i, jnp.float32(jnp.inf), dm)

    inv = 1.0 / recsum
    for k in range(3):
        idx_ref[pl.ds(k, 1), :] = idxs[k] + b * S
        w_ref[pl.ds(k, 1), :] = recs[k] * inv


def _k2_sc_body(i0_hbm, i1_hbm, i2_hbm, w0_hbm, w1_hbm, w2_hbm,
                tab_hbm, itp_hbm,
                i0_v, i1_v, i2_v, w0_v, w1_v, w2_v,
                a0_v, a1_v, a2_v, b0_v, b1_v, b2_v, out_v, sem_a, sem_b):
    wid = lax.axis_index("s") * SC_CORES + lax.axis_index("c")
    base = wid * PTS_PER_W
    idx_vs = (i0_v, i1_v, i2_v)
    w_vs = (w0_v, w1_v, w2_v)
    bufs = ((a0_v, a1_v, a2_v), (b0_v, b1_v, b2_v))
    sems = (sem_a, sem_b)
    nch = PTS_PER_W // CH

    for k, h in enumerate((i0_hbm, i1_hbm, i2_hbm)):
        pltpu.sync_copy(h.at[pl.ds(base, PTS_PER_W)], idx_vs[k])
    for k, h in enumerate((w0_hbm, w1_hbm, w2_hbm)):
        pltpu.sync_copy(h.at[pl.ds(base, PTS_PER_W)], w_vs[k])

    def fire(c, par):
        for k in range(3):
            pltpu.async_copy(
                tab_hbm.at[idx_vs[k].at[pl.ds(c * CH, CH)]], bufs[par][k],
                sems[par])

    def drain(c, par):
        for k in range(3):
            pltpu.make_async_copy(
                tab_hbm.at[idx_vs[k].at[pl.ds(c * CH, CH)]], bufs[par][k],
                sems[par]).wait()

    def compute(c, par):
        rows = bufs[par]

        def group(g, _):
            gb = g * 16
            wv = [w_vs[k][pl.ds(c * CH + gb, 16)] for k in range(3)]
            for pp in range(16):
                p = gb + pp
                for dv in range(D2 // 16):
                    sl = pl.ds(dv * 16, 16)
                    out_v[p, sl] = (rows[0][p, sl] * wv[0][pp]
                                    + rows[1][p, sl] * wv[1][pp]
                                    + rows[2][p, sl] * wv[2][pp])
            return _

        lax.fori_loop(0, CH // 16, group, None)
        pltpu.sync_copy(out_v, itp_hbm.at[pl.ds(base + c * CH, CH)])

    fire(0, 0)

    def pair(t, _):
        c0 = 2 * t
        fire(c0 + 1, 1)
        drain(c0, 0)
        compute(c0, 0)

        @pl.when(t + 1 < nch // 2)
        def _():
            fire(c0 + 2, 0)

        drain(c0 + 1, 1)
        compute(c0 + 1, 1)
        return _

    lax.fori_loop(0, nch // 2, pair, None)


def _mlp_body(p1_ref, itp_ref, w0a_ref, w0b_ref, b0_ref, w1_ref, b1_ref,
              g0_ref, be0_ref, g1_ref, be1_ref, out_ref,
              hbuf, s0_ref, ss0_ref, s1_ref, ss1_ref):
    ph = pl.program_id(0)
    b = pl.program_id(1)
    j = pl.program_id(2)
    n = jnp.float32(B * N)
    sl = pl.ds(j * NBF, NBF)

    @pl.when(ph == 0)
    def _phase0():
        @pl.when((b == 0) & (j == 0))
        def _init():
            s0_ref[...] = jnp.zeros_like(s0_ref)
            ss0_ref[...] = jnp.zeros_like(ss0_ref)

        p1 = p1_ref[0]
        itp = itp_ref[...]
        h0 = jax.lax.dot_general(w0a_ref[...], p1, (((1,), (0,)), ((), ())),
                                 preferred_element_type=jnp.float32)
        h0 = h0 + jax.lax.dot_general(
            w0b_ref[...], itp, (((1,), (1,)), ((), ())),
            preferred_element_type=jnp.float32)
        h0 = h0 + b0_ref[...]
        hbuf[b, :, sl] = h0
        s0_ref[...] += jnp.sum(h0, axis=1, keepdims=True)
        ss0_ref[...] += jnp.sum(h0 * h0, axis=1, keepdims=True)

    @pl.when(ph == 1)
    def _phase1():
        @pl.when((b == 0) & (j == 0))
        def _init():
            s1_ref[...] = jnp.zeros_like(s1_ref)
            ss1_ref[...] = jnp.zeros_like(ss1_ref)

        mean0 = s0_ref[...] / n
        var0 = ss0_ref[...] / n - mean0 * mean0
        a0 = g0_ref[...] * jax.lax.rsqrt(var0 + EPS)
        c0 = be0_ref[...] - mean0 * a0
        z = jnp.maximum(hbuf[b, :, sl] * a0 + c0, 0.0)
        h1 = jax.lax.dot_general(w1_ref[...], z, (((1,), (0,)), ((), ())),
                                 preferred_element_type=jnp.float32)
        h1 = h1 + b1_ref[...]
        hbuf[b, :, sl] = h1
        s1_ref[...] += jnp.sum(h1, axis=1, keepdims=True)
        ss1_ref[...] += jnp.sum(h1 * h1, axis=1, keepdims=True)

    @pl.when(ph == 2)
    def _phase2():
        mean1 = s1_ref[...] / n
        var1 = ss1_ref[...] / n - mean1 * mean1
        a1 = g1_ref[...] * jax.lax.rsqrt(var1 + EPS)
        c1 = be1_ref[...] - mean1 * a1
        out_ref[0] = jnp.maximum(hbuf[b, :, sl] * a1 + c1, 0.0)


def _run_topk(xyz1, xyz2):
    f32 = jnp.float32
    x2m = xyz2 * jnp.float32(-2.0)
    sq1 = jnp.sum(xyz1 * xyz1, axis=1, keepdims=True)
    sq2 = jnp.sum(xyz2 * xyz2, axis=1)[:, :, None]
    nbs = N // NB1
    gidx, wts = pl.pallas_call(
        _k1_body,
        grid=(B, nbs),
        in_specs=[
            pl.BlockSpec((1, 3, NB1), lambda b, j: (b, 0, j)),
            pl.BlockSpec((1, 3, S), lambda b, j: (b, 0, 0)),
            pl.BlockSpec((1, 1, NB1), lambda b, j: (b, 0, j)),
            pl.BlockSpec((1, S, 1), lambda b, j: (b, 0, 0)),
        ],
        out_specs=[
            pl.BlockSpec((3, NB1), lambda b, j: (0, b * (N // NB1) + j)),
            pl.BlockSpec((3, NB1), lambda b, j: (0, b * (N // NB1) + j)),
        ],
        out_shape=[
            jax.ShapeDtypeStruct((3, B * N), jnp.int32),
            jax.ShapeDtypeStruct((3, B * N), f32),
        ],
    )(xyz1, x2m, sq1, sq2)
    return gidx, wts


def _run_sc_interp(gidx_f, wts_f, points2):
    f32 = jnp.float32
    table = jnp.transpose(points2, (0, 2, 1)).reshape(B * S, D2)

    sc_gather = pl.kernel(
        _k2_sc_body,
        out_type=jax.ShapeDtypeStruct((B * N, D2), f32),
        mesh=plsc.VectorSubcoreMesh(core_axis_name="c", subcore_axis_name="s"),
        scratch_types=[
            pltpu.VMEM((PTS_PER_W,), jnp.int32),
            pltpu.VMEM((PTS_PER_W,), jnp.int32),
            pltpu.VMEM((PTS_PER_W,), jnp.int32),
            pltpu.VMEM((PTS_PER_W,), f32),
            pltpu.VMEM((PTS_PER_W,), f32),
            pltpu.VMEM((PTS_PER_W,), f32),
            pltpu.VMEM((CH, D2), f32),
            pltpu.VMEM((CH, D2), f32),
            pltpu.VMEM((CH, D2), f32),
            pltpu.VMEM((CH, D2), f32),
            pltpu.VMEM((CH, D2), f32),
            pltpu.VMEM((CH, D2), f32),
            pltpu.VMEM((CH, D2), f32),
            pltpu.SemaphoreType.DMA,
            pltpu.SemaphoreType.DMA,
        ],
    )
    itp = sc_gather(gidx_f[0], gidx_f[1], gidx_f[2],
                    wts_f[0], wts_f[1], wts_f[2], table)
    return itp


def kernel(xyz1, xyz2, points1, points2, w0, b0, g0, be0, w1, b1, g1, be1):
    f32 = jnp.float32
    w0a = w0[:, :D1]
    w0b = w0[:, D1:]
    col = lambda v: v.reshape(-1, 1).astype(f32)

    gidx_f, wts_f = _run_topk(xyz1, xyz2)
    itp = _run_sc_interp(gidx_f, wts_f, points2)

    nj = N // NBF
    out = pl.pallas_call(
        _mlp_body,
        grid=(3, B, nj),
        in_specs=[
            pl.BlockSpec((1, D1, NBF),
                         lambda ph, b, j: (jnp.where(ph == 0, b, 0), 0,
                                           jnp.where(ph == 0, j, 0))),
            pl.BlockSpec((NBF, D2),
                         lambda ph, b, j: (jnp.where(ph == 0, b * nj + j, 0),
                                           0)),
            pl.BlockSpec((C0, D1), lambda ph, b, j: (0, 0)),
            pl.BlockSpec((C0, D2), lambda ph, b, j: (0, 0)),
            pl.BlockSpec((C0, 1), lambda ph, b, j: (0, 0)),
            pl.BlockSpec((C1, C0), lambda ph, b, j: (0, 0)),
            pl.BlockSpec((C1, 1), lambda ph, b, j: (0, 0)),
            pl.BlockSpec((C0, 1), lambda ph, b, j: (0, 0)),
            pl.BlockSpec((C0, 1), lambda ph, b, j: (0, 0)),
            pl.BlockSpec((C1, 1), lambda ph, b, j: (0, 0)),
            pl.BlockSpec((C1, 1), lambda ph, b, j: (0, 0)),
        ],
        out_specs=pl.BlockSpec(
            (1, C1, NBF),
            lambda ph, b, j: (jnp.where(ph == 2, b, 0), 0,
                              jnp.where(ph == 2, j, 0))),
        out_shape=jax.ShapeDtypeStruct((B, C1, N), f32),
        scratch_shapes=[
            pltpu.VMEM((B, C0, N), f32),
            pltpu.VMEM((C0, 1), f32),
            pltpu.VMEM((C0, 1), f32),
            pltpu.VMEM((C1, 1), f32),
            pltpu.VMEM((C1, 1), f32),
        ],
    )(points1, itp, w0a, w0b, col(b0), w1, col(b1),
      col(g0), col(be0), col(g1), col(be1))

    return out


# speedup vs baseline: 1.5175x; 1.0698x over previous
import functools

import jax
import jax.numpy as jnp
from jax import lax
from jax.experimental import pallas as pl
from jax.experimental.pallas import tpu as pltpu
from jax.experimental.pallas import tpu_sc as plsc

B, N, S, D1, D2 = 4, 8192, 2048, 64, 128
C0, C1 = 128, 128
NB1 = 4096
NBF = 2048
EPS = 1e-5

SC_CORES, SC_SUBCORES = 2, 16
NW = SC_CORES * SC_SUBCORES
PTS_PER_W = (B * N) // NW
CH = 64


def _k1_body(x1_ref, x2m_ref, sq1_ref, sq2_ref, idx_ref, w_ref):
    b = pl.program_id(0)
    x1 = x1_ref[0]
    x2m = x2m_ref[0]
    sq1r = sq1_ref[0]
    sq2c = sq2_ref[0]

    dm = jax.lax.dot_general(x2m, x1, (((0,), (0,)), ((), ())),
                             preferred_element_type=jnp.float32) + sq2c

    iota = jax.lax.broadcasted_iota(jnp.int32, dm.shape, 0)
    idxs, recs = [], []
    recsum = jnp.zeros((1, dm.shape[1]), jnp.float32)
    for k in range(3):
        m = jnp.min(dm, axis=0, keepdims=True)
        i = jnp.min(jnp.where(dm == m, iota, S), axis=0, keepdims=True)
        rec = 1.0 / ((m + sq1r) + 1e-8)
        idxs.append(i)
        recs.append(rec)
        recsum = recsum + rec
        if k < 2:
            dm = jnp.where(iota == i, jnp.float32(jnp.inf), dm)

    inv = 1.0 / recsum
    for k in range(3):
        idx_ref[pl.ds(k, 1), :] = idxs[k] + b * S
        w_ref[pl.ds(k, 1), :] = recs[k] * inv


def _k2_sc_body(i0_hbm, i1_hbm, i2_hbm, w0_hbm, w1_hbm, w2_hbm,
                tab_hbm, itp_hbm,
                i0_v, i1_v, i2_v, w0_v, w1_v, w2_v,
                a0_v, a1_v, a2_v, b0_v, b1_v, b2_v, out_v, sem_a, sem_b):
    wid = lax.axis_index("s") * SC_CORES + lax.axis_index("c")
    base = wid * PTS_PER_W
    idx_vs = (i0_v, i1_v, i2_v)
    w_vs = (w0_v, w1_v, w2_v)
    bufs = ((a0_v, a1_v, a2_v), (b0_v, b1_v, b2_v))
    sems = (sem_a, sem_b)
    nch = PTS_PER_W // CH

    for k, h in enumerate((i0_hbm, i1_hbm, i2_hbm)):
        pltpu.sync_copy(h.at[pl.ds(base, PTS_PER_W)], idx_vs[k])
    for k, h in enumerate((w0_hbm, w1_hbm, w2_hbm)):
        pltpu.sync_copy(h.at[pl.ds(base, PTS_PER_W)], w_vs[k])

    def fire(c, par):
        for k in range(3):
            pltpu.async_copy(
                tab_hbm.at[idx_vs[k].at[pl.ds(c * CH, CH)]], bufs[par][k],
                sems[par])

    def drain(c, par):
        for k in range(3):
            pltpu.make_async_copy(
                tab_hbm.at[idx_vs[k].at[pl.ds(c * CH, CH)]], bufs[par][k],
                sems[par]).wait()

    def compute(c, par):
        rows = bufs[par]

        def group(g, _):
            gb = g * 16
            wv = [w_vs[k][pl.ds(c * CH + gb, 16)] for k in range(3)]
            for pp in range(16):
                p = gb + pp
                for dv in range(D2 // 16):
                    sl = pl.ds(dv * 16, 16)
                    out_v[p, sl] = (rows[0][p, sl] * wv[0][pp]
                                    + rows[1][p, sl] * wv[1][pp]
                                    + rows[2][p, sl] * wv[2][pp])
            return _

        lax.fori_loop(0, CH // 16, group, None)
        pltpu.sync_copy(out_v, itp_hbm.at[pl.ds(base + c * CH, CH)])

    fire(0, 0)

    def pair(t, _):
        c0 = 2 * t
        fire(c0 + 1, 1)
        drain(c0, 0)
        compute(c0, 0)

        @pl.when(t + 1 < nch // 2)
        def _():
            fire(c0 + 2, 0)

        drain(c0 + 1, 1)
        compute(c0 + 1, 1)
        return _

    lax.fori_loop(0, nch // 2, pair, None)


def _mlp_body(p1_ref, itp_ref, w0a_ref, w0b_ref, b0_ref, w1_ref, b1_ref,
              g0_ref, be0_ref, g1_ref, be1_ref, out_ref,
              hbuf, s0_ref, ss0_ref, s1_ref, ss1_ref):
    ph = pl.program_id(0)
    b = pl.program_id(1)
    j = pl.program_id(2)
    n = jnp.float32(B * N)
    sl = pl.ds(j * NBF, NBF)

    @pl.when(ph == 0)
    def _phase0():
        @pl.when((b == 0) & (j == 0))
        def _init():
            s0_ref[...] = jnp.zeros_like(s0_ref)
            ss0_ref[...] = jnp.zeros_like(ss0_ref)

        p1 = p1_ref[0]
        itp = itp_ref[...]
        h0 = jax.lax.dot_general(w0a_ref[...], p1, (((1,), (0,)), ((), ())),
                                 preferred_element_type=jnp.float32)
        h0 = h0 + jax.lax.dot_general(
            w0b_ref[...], itp, (((1,), (1,)), ((), ())),
            preferred_element_type=jnp.float32)
        h0 = h0 + b0_ref[...]
        hbuf[b, :, sl] = h0
        s0_ref[...] += jnp.sum(h0, axis=1, keepdims=True)
        ss0_ref[...] += jnp.sum(h0 * h0, axis=1, keepdims=True)

    @pl.when(ph == 1)
    def _phase1():
        @pl.when((b == 0) & (j == 0))
        def _init():
            s1_ref[...] = jnp.zeros_like(s1_ref)
            ss1_ref[...] = jnp.zeros_like(ss1_ref)

        mean0 = s0_ref[...] / n
        var0 = ss0_ref[...] / n - mean0 * mean0
        a0 = g0_ref[...] * jax.lax.rsqrt(var0 + EPS)
        c0 = be0_ref[...] - mean0 * a0
        z = jnp.maximum(hbuf[b, :, sl] * a0 + c0, 0.0)
        h1 = jax.lax.dot_general(w1_ref[...], z, (((1,), (0,)), ((), ())),
                                 preferred_element_type=jnp.float32)
        h1 = h1 + b1_ref[...]
        hbuf[b, :, sl] = h1
        s1_ref[...] += jnp.sum(h1, axis=1, keepdims=True)
        ss1_ref[...] += jnp.sum(h1 * h1, axis=1, keepdims=True)

    @pl.when(ph == 2)
    def _phase2():
        mean1 = s1_ref[...] / n
        var1 = ss1_ref[...] / n - mean1 * mean1
        a1 = g1_ref[...] * jax.lax.rsqrt(var1 + EPS)
        c1 = be1_ref[...] - mean1 * a1
        out_ref[0] = jnp.maximum(hbuf[b, :, sl] * a1 + c1, 0.0)


def _run_topk(xyz1, xyz2):
    f32 = jnp.float32
    x2m = xyz2 * jnp.float32(-2.0)
    sq1 = jnp.sum(xyz1 * xyz1, axis=1, keepdims=True)
    sq2 = jnp.sum(xyz2 * xyz2, axis=1)[:, :, None]
    nbs = N // NB1
    gidx, wts = pl.pallas_call(
        _k1_body,
        grid=(B, nbs),
        in_specs=[
            pl.BlockSpec((1, 3, NB1), lambda b, j: (b, 0, j)),
            pl.BlockSpec((1, 3, S), lambda b, j: (b, 0, 0)),
            pl.BlockSpec((1, 1, NB1), lambda b, j: (b, 0, j)),
            pl.BlockSpec((1, S, 1), lambda b, j: (b, 0, 0)),
        ],
        out_specs=[
            pl.BlockSpec((3, NB1), lambda b, j: (0, b * (N // NB1) + j)),
            pl.BlockSpec((3, NB1), lambda b, j: (0, b * (N // NB1) + j)),
        ],
        out_shape=[
            jax.ShapeDtypeStruct((3, B * N), jnp.int32),
            jax.ShapeDtypeStruct((3, B * N), f32),
        ],
    )(xyz1, x2m, sq1, sq2)
    return gidx, wts


def _run_sc_interp(gidx_f, wts_f, points2):
    f32 = jnp.float32
    table = jnp.transpose(points2, (0, 2, 1)).reshape(B * S, D2)

    sc_gather = pl.kernel(
        _k2_sc_body,
        out_type=jax.ShapeDtypeStruct((B * N, D2), f32),
        mesh=plsc.VectorSubcoreMesh(core_axis_name="c", subcore_axis_name="s"),
        scratch_types=[
            pltpu.VMEM((PTS_PER_W,), jnp.int32),
            pltpu.VMEM((PTS_PER_W,), jnp.int32),
            pltpu.VMEM((PTS_PER_W,), jnp.int32),
            pltpu.VMEM((PTS_PER_W,), f32),
            pltpu.VMEM((PTS_PER_W,), f32),
            pltpu.VMEM((PTS_PER_W,), f32),
            pltpu.VMEM((CH, D2), f32),
            pltpu.VMEM((CH, D2), f32),
            pltpu.VMEM((CH, D2), f32),
            pltpu.VMEM((CH, D2), f32),
            pltpu.VMEM((CH, D2), f32),
            pltpu.VMEM((CH, D2), f32),
            pltpu.VMEM((CH, D2), f32),
            pltpu.SemaphoreType.DMA,
            pltpu.SemaphoreType.DMA,
        ],
    )
    itp = sc_gather(gidx_f[0], gidx_f[1], gidx_f[2],
                    wts_f[0], wts_f[1], wts_f[2], table)
    return itp


def kernel(xyz1, xyz2, points1, points2, w0, b0, g0, be0, w1, b1, g1, be1):
    f32 = jnp.float32
    w0a = w0[:, :D1]
    w0b = w0[:, D1:]
    col = lambda v: v.reshape(-1, 1).astype(f32)

    gidx_f, wts_f = _run_topk(xyz1, xyz2)
    itp = _run_sc_interp(gidx_f, wts_f, points2)

    nj = N // NBF
    out = pl.pallas_call(
        _mlp_body,
        grid=(3, B, nj),
        in_specs=[
            pl.BlockSpec((1, D1, NBF),
                         lambda ph, b, j: (jnp.where(ph == 0, b, 0), 0,
                                           jnp.where(ph == 0, j, 0))),
            pl.BlockSpec((NBF, D2),
                         lambda ph, b, j: (jnp.where(ph == 0, b * nj + j, 0),
                                           0)),
            pl.BlockSpec((C0, D1), lambda ph, b, j: (0, 0)),
            pl.BlockSpec((C0, D2), lambda ph, b, j: (0, 0)),
            pl.BlockSpec((C0, 1), lambda ph, b, j: (0, 0)),
            pl.BlockSpec((C1, C0), lambda ph, b, j: (0, 0)),
            pl.BlockSpec((C1, 1), lambda ph, b, j: (0, 0)),
            pl.BlockSpec((C0, 1), lambda ph, b, j: (0, 0)),
            pl.BlockSpec((C0, 1), lambda ph, b, j: (0, 0)),
            pl.BlockSpec((C1, 1), lambda ph, b, j: (0, 0)),
            pl.BlockSpec((C1, 1), lambda ph, b, j: (0, 0)),
        ],
        out_specs=pl.BlockSpec(
            (1, C1, NBF),
            lambda ph, b, j: (jnp.where(ph == 2, b, 0), 0,
                              jnp.where(ph == 2, j, 0))),
        out_shape=jax.ShapeDtypeStruct((B, C1, N), f32),
        scratch_shapes=[
            pltpu.VMEM((B, C0, N), f32),
            pltpu.VMEM((C0, 1), f32),
            pltpu.VMEM((C0, 1), f32),
            pltpu.VMEM((C1, 1), f32),
            pltpu.VMEM((C1, 1), f32),
        ],
    )(points1, itp, w0a, w0b, col(b0), w1, col(b1),
      col(g0), col(be0), col(g1), col(be1))

    return out


# speedup vs baseline: 1.5636x; 1.0304x over previous
import functools

import jax
import jax.numpy as jnp
from jax import lax
from jax.experimental import pallas as pl
from jax.experimental.pallas import tpu as pltpu
from jax.experimental.pallas import tpu_sc as plsc

B, N, S, D1, D2 = 4, 8192, 2048, 64, 128
C0, C1 = 128, 128
NB1 = 4096
NBF = 4096
EPS = 1e-5

SC_CORES, SC_SUBCORES = 2, 16
NW = SC_CORES * SC_SUBCORES
PTS_PER_W = (B * N) // NW
CH = 64


def _k1_body(x1_ref, x2m_ref, sq1_ref, sq2_ref, idx_ref, w_ref):
    b = pl.program_id(0)
    x1 = x1_ref[0]
    x2m = x2m_ref[0]
    sq1r = sq1_ref[0]
    sq2c = sq2_ref[0]

    dm = jax.lax.dot_general(x2m, x1, (((0,), (0,)), ((), ())),
                             preferred_element_type=jnp.float32) + sq2c

    iota = jax.lax.broadcasted_iota(jnp.int32, dm.shape, 0)
    idxs, recs = [], []
    recsum = jnp.zeros((1, dm.shape[1]), jnp.float32)
    for k in range(3):
        m = jnp.min(dm, axis=0, keepdims=True)
        i = jnp.min(jnp.where(dm == m, iota, S), axis=0, keepdims=True)
        rec = 1.0 / ((m + sq1r) + 1e-8)
        idxs.append(i)
        recs.append(rec)
        recsum = recsum + rec
        if k < 2:
            dm = jnp.where(iota == i, jnp.float32(jnp.inf), dm)

    inv = 1.0 / recsum
    for k in range(3):
        idx_ref[pl.ds(k, 1), :] = idxs[k] + b * S
        w_ref[pl.ds(k, 1), :] = recs[k] * inv


def _k2_sc_body(i0_hbm, i1_hbm, i2_hbm, w0_hbm, w1_hbm, w2_hbm,
                tab_hbm, itp_hbm,
                i0_v, i1_v, i2_v, w0_v, w1_v, w2_v,
                a0_v, a1_v, a2_v, b0_v, b1_v, b2_v, out_v, sem_a, sem_b):
    wid = lax.axis_index("s") * SC_CORES + lax.axis_index("c")
    base = wid * PTS_PER_W
    idx_vs = (i0_v, i1_v, i2_v)
    w_vs = (w0_v, w1_v, w2_v)
    bufs = ((a0_v, a1_v, a2_v), (b0_v, b1_v, b2_v))
    sems = (sem_a, sem_b)
    nch = PTS_PER_W // CH

    for k, h in enumerate((i0_hbm, i1_hbm, i2_hbm)):
        pltpu.sync_copy(h.at[pl.ds(base, PTS_PER_W)], idx_vs[k])
    for k, h in enumerate((w0_hbm, w1_hbm, w2_hbm)):
        pltpu.sync_copy(h.at[pl.ds(base, PTS_PER_W)], w_vs[k])

    def fire(c, par):
        for k in range(3):
            pltpu.async_copy(
                tab_hbm.at[idx_vs[k].at[pl.ds(c * CH, CH)]], bufs[par][k],
                sems[par])

    def drain(c, par):
        for k in range(3):
            pltpu.make_async_copy(
                tab_hbm.at[idx_vs[k].at[pl.ds(c * CH, CH)]], bufs[par][k],
                sems[par]).wait()

    def compute(c, par):
        rows = bufs[par]

        def group(g, _):
            gb = g * 16
            wv = [w_vs[k][pl.ds(c * CH + gb, 16)] for k in range(3)]
            for pp in range(16):
                p = gb + pp
                for dv in range(D2 // 16):
                    sl = pl.ds(dv * 16, 16)
                    out_v[p, sl] = (rows[0][p, sl] * wv[0][pp]
                                    + rows[1][p, sl] * wv[1][pp]
                                    + rows[2][p, sl] * wv[2][pp])
            return _

        lax.fori_loop(0, CH // 16, group, None)
        pltpu.sync_copy(out_v, itp_hbm.at[pl.ds(base + c * CH, CH)])

    fire(0, 0)

    def pair(t, _):
        c0 = 2 * t
        fire(c0 + 1, 1)
        drain(c0, 0)
        compute(c0, 0)

        @pl.when(t + 1 < nch // 2)
        def _():
            fire(c0 + 2, 0)

        drain(c0 + 1, 1)
        compute(c0 + 1, 1)
        return _

    lax.fori_loop(0, nch // 2, pair, None)


def _mlp_body(p1_ref, itp_ref, w0a_ref, w0b_ref, b0_ref, w1_ref, b1_ref,
              g0_ref, be0_ref, g1_ref, be1_ref, out_ref,
              hbuf, s0_ref, ss0_ref, s1_ref, ss1_ref):
    ph = pl.program_id(0)
    b = pl.program_id(1)
    j = pl.program_id(2)
    n = jnp.float32(B * N)
    sl = pl.ds(j * NBF, NBF)

    @pl.when(ph == 0)
    def _phase0():
        @pl.when((b == 0) & (j == 0))
        def _init():
            s0_ref[...] = jnp.zeros_like(s0_ref)
            ss0_ref[...] = jnp.zeros_like(ss0_ref)

        p1 = p1_ref[0]
        itp = itp_ref[...]
        h0 = jax.lax.dot_general(w0a_ref[...], p1, (((1,), (0,)), ((), ())),
                                 preferred_element_type=jnp.float32)
        h0 = h0 + jax.lax.dot_general(
            w0b_ref[...], itp, (((1,), (1,)), ((), ())),
            preferred_element_type=jnp.float32)
        h0 = h0 + b0_ref[...]
        hbuf[b, :, sl] = h0
        s0_ref[...] += jnp.sum(h0, axis=1, keepdims=True)
        ss0_ref[...] += jnp.sum(h0 * h0, axis=1, keepdims=True)

    @pl.when(ph == 1)
    def _phase1():
        @pl.when((b == 0) & (j == 0))
        def _init():
            s1_ref[...] = jnp.zeros_like(s1_ref)
            ss1_ref[...] = jnp.zeros_like(ss1_ref)

        mean0 = s0_ref[...] / n
        var0 = ss0_ref[...] / n - mean0 * mean0
        a0 = g0_ref[...] * jax.lax.rsqrt(var0 + EPS)
        c0 = be0_ref[...] - mean0 * a0
        z = jnp.maximum(hbuf[b, :, sl] * a0 + c0, 0.0)
        h1 = jax.lax.dot_general(w1_ref[...], z, (((1,), (0,)), ((), ())),
                                 preferred_element_type=jnp.float32)
        h1 = h1 + b1_ref[...]
        hbuf[b, :, sl] = h1
        s1_ref[...] += jnp.sum(h1, axis=1, keepdims=True)
        ss1_ref[...] += jnp.sum(h1 * h1, axis=1, keepdims=True)

    @pl.when(ph == 2)
    def _phase2():
        mean1 = s1_ref[...] / n
        var1 = ss1_ref[...] / n - mean1 * mean1
        a1 = g1_ref[...] * jax.lax.rsqrt(var1 + EPS)
        c1 = be1_ref[...] - mean1 * a1
        out_ref[0] = jnp.maximum(hbuf[b, :, sl] * a1 + c1, 0.0)


def _run_topk(xyz1, xyz2):
    f32 = jnp.float32
    x2m = xyz2 * jnp.float32(-2.0)
    sq1 = jnp.sum(xyz1 * xyz1, axis=1, keepdims=True)
    sq2 = jnp.sum(xyz2 * xyz2, axis=1)[:, :, None]
    nbs = N // NB1
    gidx, wts = pl.pallas_call(
        _k1_body,
        grid=(B, nbs),
        in_specs=[
            pl.BlockSpec((1, 3, NB1), lambda b, j: (b, 0, j)),
            pl.BlockSpec((1, 3, S), lambda b, j: (b, 0, 0)),
            pl.BlockSpec((1, 1, NB1), lambda b, j: (b, 0, j)),
            pl.BlockSpec((1, S, 1), lambda b, j: (b, 0, 0)),
        ],
        out_specs=[
            pl.BlockSpec((3, NB1), lambda b, j: (0, b * (N // NB1) + j)),
            pl.BlockSpec((3, NB1), lambda b, j: (0, b * (N // NB1) + j)),
        ],
        out_shape=[
            jax.ShapeDtypeStruct((3, B * N), jnp.int32),
            jax.ShapeDtypeStruct((3, B * N), f32),
        ],
    )(xyz1, x2m, sq1, sq2)
    return gidx, wts


def _run_sc_interp(gidx_f, wts_f, points2):
    f32 = jnp.float32
    table = jnp.transpose(points2, (0, 2, 1)).reshape(B * S, D2)

    sc_gather = pl.kernel(
        _k2_sc_body,
        out_type=jax.ShapeDtypeStruct((B * N, D2), f32),
        mesh=plsc.VectorSubcoreMesh(core_axis_name="c", subcore_axis_name="s"),
        scratch_types=[
            pltpu.VMEM((PTS_PER_W,), jnp.int32),
            pltpu.VMEM((PTS_PER_W,), jnp.int32),
            pltpu.VMEM((PTS_PER_W,), jnp.int32),
            pltpu.VMEM((PTS_PER_W,), f32),
            pltpu.VMEM((PTS_PER_W,), f32),
            pltpu.VMEM((PTS_PER_W,), f32),
            pltpu.VMEM((CH, D2), f32),
            pltpu.VMEM((CH, D2), f32),
            pltpu.VMEM((CH, D2), f32),
            pltpu.VMEM((CH, D2), f32),
            pltpu.VMEM((CH, D2), f32),
            pltpu.VMEM((CH, D2), f32),
            pltpu.VMEM((CH, D2), f32),
            pltpu.SemaphoreType.DMA,
            pltpu.SemaphoreType.DMA,
        ],
    )
    itp = sc_gather(gidx_f[0], gidx_f[1], gidx_f[2],
                    wts_f[0], wts_f[1], wts_f[2], table)
    return itp


def kernel(xyz1, xyz2, points1, points2, w0, b0, g0, be0, w1, b1, g1, be1):
    f32 = jnp.float32
    w0a = w0[:, :D1]
    w0b = w0[:, D1:]
    col = lambda v: v.reshape(-1, 1).astype(f32)

    gidx_f, wts_f = _run_topk(xyz1, xyz2)
    itp = _run_sc_interp(gidx_f, wts_f, points2)

    nj = N // NBF
    out = pl.pallas_call(
        _mlp_body,
        grid=(3, B, nj),
        in_specs=[
            pl.BlockSpec((1, D1, NBF),
                         lambda ph, b, j: (jnp.where(ph == 0, b, 0), 0,
                                           jnp.where(ph == 0, j, 0))),
            pl.BlockSpec((NBF, D2),
                         lambda ph, b, j: (jnp.where(ph == 0, b * nj + j, 0),
                                           0)),
            pl.BlockSpec((C0, D1), lambda ph, b, j: (0, 0)),
            pl.BlockSpec((C0, D2), lambda ph, b, j: (0, 0)),
            pl.BlockSpec((C0, 1), lambda ph, b, j: (0, 0)),
            pl.BlockSpec((C1, C0), lambda ph, b, j: (0, 0)),
            pl.BlockSpec((C1, 1), lambda ph, b, j: (0, 0)),
            pl.BlockSpec((C0, 1), lambda ph, b, j: (0, 0)),
            pl.BlockSpec((C0, 1), lambda ph, b, j: (0, 0)),
            pl.BlockSpec((C1, 1), lambda ph, b, j: (0, 0)),
            pl.BlockSpec((C1, 1), lambda ph, b, j: (0, 0)),
        ],
        out_specs=pl.BlockSpec(
            (1, C1, NBF),
            lambda ph, b, j: (jnp.where(ph == 2, b, 0), 0,
                              jnp.where(ph == 2, j, 0))),
        out_shape=jax.ShapeDtypeStruct((B, C1, N), f32),
        scratch_shapes=[
            pltpu.VMEM((B, C0, N), f32),
            pltpu.VMEM((C0, 1), f32),
            pltpu.VMEM((C0, 1), f32),
            pltpu.VMEM((C1, 1), f32),
            pltpu.VMEM((C1, 1), f32),
        ],
    )(points1, itp, w0a, w0b, col(b0), w1, col(b1),
      col(g0), col(be0), col(g1), col(be1))

    return out


# speedup vs baseline: 1.5823x; 1.0120x over previous
import functools

import jax
import jax.numpy as jnp
from jax import lax
from jax.experimental import pallas as pl
from jax.experimental.pallas import tpu as pltpu
from jax.experimental.pallas import tpu_sc as plsc

B, N, S, D1, D2 = 4, 8192, 2048, 64, 128
C0, C1 = 128, 128
NB1 = 4096
NBF = 8192
EPS = 1e-5

SC_CORES, SC_SUBCORES = 2, 16
NW = SC_CORES * SC_SUBCORES
PTS_PER_W = (B * N) // NW
CH = 64


def _k1_body(x1_ref, x2m_ref, sq1_ref, sq2_ref, idx_ref, w_ref):
    b = pl.program_id(0)
    x1 = x1_ref[0]
    x2m = x2m_ref[0]
    sq1r = sq1_ref[0]
    sq2c = sq2_ref[0]

    dm = jax.lax.dot_general(x2m, x1, (((0,), (0,)), ((), ())),
                             preferred_element_type=jnp.float32) + sq2c

    iota = jax.lax.broadcasted_iota(jnp.int32, dm.shape, 0)
    idxs, recs = [], []
    recsum = jnp.zeros((1, dm.shape[1]), jnp.float32)
    for k in range(3):
        m = jnp.min(dm, axis=0, keepdims=True)
        i = jnp.min(jnp.where(dm == m, iota, S), axis=0, keepdims=True)
        rec = 1.0 / ((m + sq1r) + 1e-8)
        idxs.append(i)
        recs.append(rec)
        recsum = recsum + rec
        if k < 2:
            dm = jnp.where(iota == i, jnp.float32(jnp.inf), dm)

    inv = 1.0 / recsum
    for k in range(3):
        idx_ref[pl.ds(k, 1), :] = idxs[k] + b * S
        w_ref[pl.ds(k, 1), :] = recs[k] * inv


def _k2_sc_body(i0_hbm, i1_hbm, i2_hbm, w0_hbm, w1_hbm, w2_hbm,
                tab_hbm, itp_hbm,
                i0_v, i1_v, i2_v, w0_v, w1_v, w2_v,
                a0_v, a1_v, a2_v, b0_v, b1_v, b2_v, out_v, sem_a, sem_b):
    wid = lax.axis_index("s") * SC_CORES + lax.axis_index("c")
    base = wid * PTS_PER_W
    idx_vs = (i0_v, i1_v, i2_v)
    w_vs = (w0_v, w1_v, w2_v)
    bufs = ((a0_v, a1_v, a2_v), (b0_v, b1_v, b2_v))
    sems = (sem_a, sem_b)
    nch = PTS_PER_W // CH

    for k, h in enumerate((i0_hbm, i1_hbm, i2_hbm)):
        pltpu.sync_copy(h.at[pl.ds(base, PTS_PER_W)], idx_vs[k])
    for k, h in enumerate((w0_hbm, w1_hbm, w2_hbm)):
        pltpu.sync_copy(h.at[pl.ds(base, PTS_PER_W)], w_vs[k])

    def fire(c, par):
        for k in range(3):
            pltpu.async_copy(
                tab_hbm.at[idx_vs[k].at[pl.ds(c * CH, CH)]], bufs[par][k],
                sems[par])

    def drain(c, par):
        for k in range(3):
            pltpu.make_async_copy(
                tab_hbm.at[idx_vs[k].at[pl.ds(c * CH, CH)]], bufs[par][k],
                sems[par]).wait()

    def compute(c, par):
        rows = bufs[par]

        def group(g, _):
            gb = g * 16
            wv = [w_vs[k][pl.ds(c * CH + gb, 16)] for k in range(3)]
            for pp in range(16):
                p = gb + pp
                for dv in range(D2 // 16):
                    sl = pl.ds(dv * 16, 16)
                    out_v[p, sl] = (rows[0][p, sl] * wv[0][pp]
                                    + rows[1][p, sl] * wv[1][pp]
                                    + rows[2][p, sl] * wv[2][pp])
            return _

        lax.fori_loop(0, CH // 16, group, None)
        pltpu.sync_copy(out_v, itp_hbm.at[pl.ds(base + c * CH, CH)])

    fire(0, 0)

    def pair(t, _):
        c0 = 2 * t
        fire(c0 + 1, 1)
        drain(c0, 0)
        compute(c0, 0)

        @pl.when(t + 1 < nch // 2)
        def _():
            fire(c0 + 2, 0)

        drain(c0 + 1, 1)
        compute(c0 + 1, 1)
        return _

    lax.fori_loop(0, nch // 2, pair, None)


def _mlp_body(p1_ref, itp_ref, w0a_ref, w0b_ref, b0_ref, w1_ref, b1_ref,
              g0_ref, be0_ref, g1_ref, be1_ref, out_ref,
              hbuf, s0_ref, ss0_ref, s1_ref, ss1_ref):
    ph = pl.program_id(0)
    b = pl.program_id(1)
    j = pl.program_id(2)
    n = jnp.float32(B * N)
    sl = pl.ds(j * NBF, NBF)

    @pl.when(ph == 0)
    def _phase0():
        @pl.when((b == 0) & (j == 0))
        def _init():
            s0_ref[...] = jnp.zeros_like(s0_ref)
            ss0_ref[...] = jnp.zeros_like(ss0_ref)

        p1 = p1_ref[0]
        itp = itp_ref[...]
        h0 = jax.lax.dot_general(w0a_ref[...], p1, (((1,), (0,)), ((), ())),
                                 preferred_element_type=jnp.float32)
        h0 = h0 + jax.lax.dot_general(
            w0b_ref[...], itp, (((1,), (1,)), ((), ())),
            preferred_element_type=jnp.float32)
        h0 = h0 + b0_ref[...]
        hbuf[b, :, sl] = h0
        s0_ref[...] += jnp.sum(h0, axis=1, keepdims=True)
        ss0_ref[...] += jnp.sum(h0 * h0, axis=1, keepdims=True)

    @pl.when(ph == 1)
    def _phase1():
        @pl.when((b == 0) & (j == 0))
        def _init():
            s1_ref[...] = jnp.zeros_like(s1_ref)
            ss1_ref[...] = jnp.zeros_like(ss1_ref)

        mean0 = s0_ref[...] / n
        var0 = ss0_ref[...] / n - mean0 * mean0
        a0 = g0_ref[...] * jax.lax.rsqrt(var0 + EPS)
        c0 = be0_ref[...] - mean0 * a0
        z = jnp.maximum(hbuf[b, :, sl] * a0 + c0, 0.0)
        h1 = jax.lax.dot_general(w1_ref[...], z, (((1,), (0,)), ((), ())),
                                 preferred_element_type=jnp.float32)
        h1 = h1 + b1_ref[...]
        hbuf[b, :, sl] = h1
        s1_ref[...] += jnp.sum(h1, axis=1, keepdims=True)
        ss1_ref[...] += jnp.sum(h1 * h1, axis=1, keepdims=True)

    @pl.when(ph == 2)
    def _phase2():
        mean1 = s1_ref[...] / n
        var1 = ss1_ref[...] / n - mean1 * mean1
        a1 = g1_ref[...] * jax.lax.rsqrt(var1 + EPS)
        c1 = be1_ref[...] - mean1 * a1
        out_ref[0] = jnp.maximum(hbuf[b, :, sl] * a1 + c1, 0.0)


def _run_topk(xyz1, xyz2):
    f32 = jnp.float32
    x2m = xyz2 * jnp.float32(-2.0)
    sq1 = jnp.sum(xyz1 * xyz1, axis=1, keepdims=True)
    sq2 = jnp.sum(xyz2 * xyz2, axis=1)[:, :, None]
    nbs = N // NB1
    gidx, wts = pl.pallas_call(
        _k1_body,
        grid=(B, nbs),
        in_specs=[
            pl.BlockSpec((1, 3, NB1), lambda b, j: (b, 0, j)),
            pl.BlockSpec((1, 3, S), lambda b, j: (b, 0, 0)),
            pl.BlockSpec((1, 1, NB1), lambda b, j: (b, 0, j)),
            pl.BlockSpec((1, S, 1), lambda b, j: (b, 0, 0)),
        ],
        out_specs=[
            pl.BlockSpec((3, NB1), lambda b, j: (0, b * (N // NB1) + j)),
            pl.BlockSpec((3, NB1), lambda b, j: (0, b * (N // NB1) + j)),
        ],
        out_shape=[
            jax.ShapeDtypeStruct((3, B * N), jnp.int32),
            jax.ShapeDtypeStruct((3, B * N), f32),
        ],
    )(xyz1, x2m, sq1, sq2)
    return gidx, wts


def _run_sc_interp(gidx_f, wts_f, points2):
    f32 = jnp.float32
    table = jnp.transpose(points2, (0, 2, 1)).reshape(B * S, D2)

    sc_gather = pl.kernel(
        _k2_sc_body,
        out_type=jax.ShapeDtypeStruct((B * N, D2), f32),
        mesh=plsc.VectorSubcoreMesh(core_axis_name="c", subcore_axis_name="s"),
        scratch_types=[
            pltpu.VMEM((PTS_PER_W,), jnp.int32),
            pltpu.VMEM((PTS_PER_W,), jnp.int32),
            pltpu.VMEM((PTS_PER_W,), jnp.int32),
            pltpu.VMEM((PTS_PER_W,), f32),
            pltpu.VMEM((PTS_PER_W,), f32),
            pltpu.VMEM((PTS_PER_W,), f32),
            pltpu.VMEM((CH, D2), f32),
            pltpu.VMEM((CH, D2), f32),
            pltpu.VMEM((CH, D2), f32),
            pltpu.VMEM((CH, D2), f32),
            pltpu.VMEM((CH, D2), f32),
            pltpu.VMEM((CH, D2), f32),
            pltpu.VMEM((CH, D2), f32),
            pltpu.SemaphoreType.DMA,
            pltpu.SemaphoreType.DMA,
        ],
    )
    itp = sc_gather(gidx_f[0], gidx_f[1], gidx_f[2],
                    wts_f[0], wts_f[1], wts_f[2], table)
    return itp


def kernel(xyz1, xyz2, points1, points2, w0, b0, g0, be0, w1, b1, g1, be1):
    f32 = jnp.float32
    w0a = w0[:, :D1]
    w0b = w0[:, D1:]
    col = lambda v: v.reshape(-1, 1).astype(f32)

    gidx_f, wts_f = _run_topk(xyz1, xyz2)
    itp = _run_sc_interp(gidx_f, wts_f, points2)

    nj = N // NBF
    out = pl.pallas_call(
        _mlp_body,
        grid=(3, B, nj),
        in_specs=[
            pl.BlockSpec((1, D1, NBF),
                         lambda ph, b, j: (jnp.where(ph == 0, b, 0), 0,
                                           jnp.where(ph == 0, j, 0))),
            pl.BlockSpec((NBF, D2),
                         lambda ph, b, j: (jnp.where(ph == 0, b * nj + j, 0),
                                           0)),
            pl.BlockSpec((C0, D1), lambda ph, b, j: (0, 0)),
            pl.BlockSpec((C0, D2), lambda ph, b, j: (0, 0)),
            pl.BlockSpec((C0, 1), lambda ph, b, j: (0, 0)),
            pl.BlockSpec((C1, C0), lambda ph, b, j: (0, 0)),
            pl.BlockSpec((C1, 1), lambda ph, b, j: (0, 0)),
            pl.BlockSpec((C0, 1), lambda ph, b, j: (0, 0)),
            pl.BlockSpec((C0, 1), lambda ph, b, j: (0, 0)),
            pl.BlockSpec((C1, 1), lambda ph, b, j: (0, 0)),
            pl.BlockSpec((C1, 1), lambda ph, b, j: (0, 0)),
        ],
        out_specs=pl.BlockSpec(
            (1, C1, NBF),
            lambda ph, b, j: (jnp.where(ph == 2, b, 0), 0,
                              jnp.where(ph == 2, j, 0))),
        out_shape=jax.ShapeDtypeStruct((B, C1, N), f32),
        scratch_shapes=[
            pltpu.VMEM((B, C0, N), f32),
            pltpu.VMEM((C0, 1), f32),
            pltpu.VMEM((C0, 1), f32),
            pltpu.VMEM((C1, 1), f32),
            pltpu.VMEM((C1, 1), f32),
        ],
    )(points1, itp, w0a, w0b, col(b0), w1, col(b1),
      col(g0), col(be0), col(g1), col(be1))

    return out
